# baseline (speedup 1.0000x reference)
import functools

import jax
import jax.numpy as jnp
from jax import lax
from jax.experimental import pallas as pl
from jax.experimental.pallas import tpu as pltpu
from jax.experimental.pallas import tpu_sc as plsc

N = 10000
E = 320000
IN_F = 128
H1_F = 128
OUT_F = 64
FH = 64

NC = 2
NS = 16
L = 16
NW = NC * NS
EPW = E // NW
C = 80
NCHUNK = EPW // C

BLK = 1024
NB = 10
N_PAD = NB * BLK
RPT = N_PAD // NS
DROW = N_PAD // L

_NEG_INF = -3.0e38


def _attn_tail(i, feat, al_ref, ar_ref, el_ref, er_ref, m_ref):
    el = jnp.sum(feat * al_ref[...][None, :], axis=1)
    er = jnp.sum(feat * ar_ref[...][None, :], axis=1)
    el_ref[0, 0, :] = el
    er_ref[0, 0, :] = er

    @pl.when(i == 0)
    def _():
        m_ref[0, 0] = _NEG_INF
        m_ref[0, 1] = _NEG_INF

    m_ref[0, 0] = jnp.maximum(m_ref[0, 0], jnp.max(el))
    m_ref[0, 1] = jnp.maximum(m_ref[0, 1], jnp.max(er))


def _feat_body(h_ref, w_ref, al_ref, ar_ref,
               fa_ref, fb_ref, el_ref, er_ref, m_ref):
    i = pl.program_id(0)
    feat = jnp.dot(h_ref[...], w_ref[...], preferred_element_type=jnp.float32)
    fa_ref[...] = feat[:, :FH]
    fb_ref[...] = feat[:, FH:]
    _attn_tail(i, feat, al_ref, ar_ref, el_ref, er_ref, m_ref)


def _make_tc_feat(F):
    return pl.pallas_call(
        _feat_body,
        grid=(NB,),
        in_specs=[
            pl.BlockSpec((BLK, IN_F), lambda i: (i, 0)),
            pl.BlockSpec((IN_F, F), lambda i: (0, 0)),
            pl.BlockSpec((F,), lambda i: (0,)),
            pl.BlockSpec((F,), lambda i: (0,)),
        ],
        out_specs=[
            pl.BlockSpec((BLK, FH), lambda i: (i, 0)),
            pl.BlockSpec((BLK, FH), lambda i: (i, 0)),
            pl.BlockSpec((1, 1, BLK), lambda i: (i, 0, 0)),
            pl.BlockSpec((1, 1, BLK), lambda i: (i, 0, 0)),
            pl.BlockSpec((1, 2), lambda i: (0, 0), memory_space=pltpu.SMEM),
        ],
        out_shape=[
            jax.ShapeDtypeStruct((N_PAD, FH), jnp.float32),
            jax.ShapeDtypeStruct((N_PAD, FH), jnp.float32),
            jax.ShapeDtypeStruct((NB, 1, BLK), jnp.float32),
            jax.ShapeDtypeStruct((NB, 1, BLK), jnp.float32),
            jax.ShapeDtypeStruct((1, 2), jnp.float32),
        ],
    )


def _den_sum(den_ref):
    return jnp.sum(den_ref[...], axis=0)


def _mid_body(aggA_ref, aggB_ref, den_ref, b_ref, w_ref, al_ref, ar_ref,
              feat_ref, el_ref, er_ref, m_ref):
    i = pl.program_id(0)
    inv = 1.0 / (_den_sum(den_ref) + 1e-16)
    hA = (aggA_ref[0] + aggA_ref[1]) * inv[:, None] + b_ref[...][None, :FH]
    hB = (aggB_ref[0] + aggB_ref[1]) * inv[:, None] + b_ref[...][None, FH:]
    h = jnp.concatenate([hA, hB], axis=1)
    h = jnp.where(h >= 0.0, h, 0.01 * h)
    feat = jnp.dot(h, w_ref[...], preferred_element_type=jnp.float32)
    feat_ref[...] = feat
    _attn_tail(i, feat, al_ref, ar_ref, el_ref, er_ref, m_ref)


def _make_tc_mid(F_in, F_out):
    return pl.pallas_call(
        _mid_body,
        grid=(NB,),
        in_specs=[
            pl.BlockSpec((NC, BLK, FH), lambda i: (0, i, 0)),
            pl.BlockSpec((NC, BLK, FH), lambda i: (0, i, 0)),
            pl.BlockSpec((NW, BLK), lambda i: (0, i)),
            pl.BlockSpec((F_in,), lambda i: (0,)),
            pl.BlockSpec((F_in, F_out), lambda i: (0, 0)),
            pl.BlockSpec((F_out,), lambda i: (0,)),
            pl.BlockSpec((F_out,), lambda i: (0,)),
        ],
        out_specs=[
            pl.BlockSpec((BLK, F_out), lambda i: (i, 0)),
            pl.BlockSpec((1, 1, BLK), lambda i: (i, 0, 0)),
            pl.BlockSpec((1, 1, BLK), lambda i: (i, 0, 0)),
            pl.BlockSpec((1, 2), lambda i: (0, 0), memory_space=pltpu.SMEM),
        ],
        out_shape=[
            jax.ShapeDtypeStruct((N_PAD, F_out), jnp.float32),
            jax.ShapeDtypeStruct((NB, 1, BLK), jnp.float32),
            jax.ShapeDtypeStruct((NB, 1, BLK), jnp.float32),
            jax.ShapeDtypeStruct((1, 2), jnp.float32),
        ],
    )


def _final_body(agg_ref, den_ref, b_ref, out_ref):
    i = pl.program_id(0)
    inv = 1.0 / (_den_sum(den_ref) + 1e-16)
    h = (agg_ref[0] + agg_ref[1]) * inv[:, None] + b_ref[...][None, :]
    h = jnp.where(h >= 0.0, h, 0.01 * h)
    row = i * BLK + lax.broadcasted_iota(jnp.int32, (BLK, 1), 0)
    h = jnp.where(row < N, h, 0.0)

    @pl.when(i == 0)
    def _():
        out_ref[...] = jnp.zeros_like(out_ref)

    out_ref[...] += jnp.sum(h, axis=0, keepdims=True)

    @pl.when(i == NB - 1)
    def _():
        out_ref[...] *= jnp.float32(1.0 / N)


def _make_tc_final(F):
    return pl.pallas_call(
        _final_body,
        grid=(NB,),
        in_specs=[
            pl.BlockSpec((NC, BLK, F), lambda i: (0, i, 0)),
            pl.BlockSpec((NW, BLK), lambda i: (0, i)),
            pl.BlockSpec((F,), lambda i: (0,)),
        ],
        out_specs=pl.BlockSpec((1, F), lambda i: (0, 0)),
        out_shape=jax.ShapeDtypeStruct((1, F), jnp.float32),
    )


def _make_sc_edge(nparts):
    mesh = plsc.VectorSubcoreMesh(core_axis_name="c", subcore_axis_name="s")

    @functools.partial(
        pl.kernel,
        out_type=(
            [jax.ShapeDtypeStruct((NC, N_PAD, FH), jnp.float32)] * nparts
            + [jax.ShapeDtypeStruct((NW, DROW, L), jnp.float32)]
        ),
        mesh=mesh,
        compiler_params=pltpu.CompilerParams(
            needs_layout_passes=False, use_tc_tiling_on_sc=False),
        scratch_types=[
            pltpu.VMEM((EPW,), jnp.int32),
            pltpu.VMEM((EPW,), jnp.int32),
            pltpu.VMEM((N_PAD,), jnp.float32),
            pltpu.VMEM((N_PAD,), jnp.float32),
            pltpu.VMEM((EPW,), jnp.float32),
            pltpu.VMEM((DROW, L), jnp.float32),
            pltpu.VMEM((L,), jnp.float32),
            pltpu.VMEM((C, FH), jnp.float32),
            pltpu.VMEM((C, FH), jnp.float32),
            pltpu.SemaphoreType.DMA,
            pltpu.SemaphoreType.DMA,
            pltpu.VMEM_SHARED((N_PAD, FH), jnp.float32),
        ],
    )
    def sc_edge(*args):
        (src1_h, dst1_h, el_h, er_h, m_h) = args[:5]
        feat_hs = args[5:5 + nparts]
        z_h = args[5 + nparts]
        agg_outs = args[6 + nparts:6 + 2 * nparts]
        den_out = args[6 + 2 * nparts]
        (src1, dst1, el_v, er_v, ee_v, den_v, m_v,
         rows_a, rows_b, gsem_a, gsem_b, agg_sh) = args[7 + 2 * nparts:]

        cid = lax.axis_index("c")
        sid = lax.axis_index("s")
        wid = cid * NS + sid
        base = sid * RPT

        pltpu.sync_copy(src1_h.at[pl.ds(wid * EPW, EPW)], src1)
        pltpu.sync_copy(dst1_h.at[pl.ds(wid * EPW, EPW)], dst1)
        pltpu.sync_copy(el_h, el_v)
        pltpu.sync_copy(er_h, er_v)
        pltpu.sync_copy(m_h, m_v)

        pltpu.sync_copy(z_h, agg_sh.at[pl.ds(base, RPT)])

        zvec = jnp.zeros((L,), jnp.float32)

        def zden(r, carry):
            den_v[r, pl.ds(0, L)] = zvec
            return carry

        lax.fori_loop(0, DROW, zden, 0)

        m_vec = m_v[...]

        def passa(t, carry):
            s_idx = src1[pl.ds(t * L, L)]
            d_idx = dst1[pl.ds(t * L, L)]
            e = plsc.load_gather(el_v, [s_idx]) + plsc.load_gather(er_v, [d_idx])
            e = jnp.where(e >= 0.0, e, 0.2 * e)
            ee = jnp.exp(e - m_vec)
            ee_v[pl.ds(t * L, L)] = ee
            plsc.addupdate_scatter(
                den_v, [lax.shift_right_logical(d_idx, 4),
                        lax.bitwise_and(d_idx, 15)], ee)
            return carry

        lax.fori_loop(0, EPW // L, passa, 0)
        pltpu.sync_copy(den_v, den_out.at[wid])

        zero16 = jnp.zeros((L,), jnp.int32)
        lane = lax.broadcasted_iota(jnp.int32, (L,), 0)

        def _scale(buf, j):
            jbase = j * C

            def scale_blk(r16, c2):
                r = r16 * L
                ee16 = ee_v[pl.ds(jbase + r, L)]
                row16 = lane + r
                for k in range(FH):
                    col = plsc.load_gather(buf, [row16, zero16 + k])
                    plsc.store_scatter(buf, [row16, zero16 + k], col * ee16)
                return c2

            lax.fori_loop(0, C // L, scale_blk, 0)

        for p in range(nparts):
            plsc.subcore_barrier()

            feat_h = feat_hs[p]
            pltpu.async_copy(
                feat_h.at[src1.at[pl.ds(0, C)]], rows_a, gsem_a)

            def passb(i, carry):
                j = 2 * i
                pltpu.async_copy(
                    feat_h.at[src1.at[pl.ds((j + 1) * C, C)]], rows_b, gsem_b)
                pltpu.make_async_copy(
                    feat_h.at[src1.at[pl.ds(j * C, C)]], rows_a, gsem_a).wait()
                _scale(rows_a, j)
                pltpu.sync_copy(rows_a,
                                agg_sh.at[dst1.at[pl.ds(j * C, C)]], add=True)

                @pl.when(j + 2 < NCHUNK)
                def _():
                    pltpu.async_copy(
                        feat_h.at[src1.at[pl.ds((j + 2) * C, C)]],
                        rows_a, gsem_a)

                pltpu.make_async_copy(
                    feat_h.at[src1.at[pl.ds((j + 1) * C, C)]],
                    rows_b, gsem_b).wait()
                _scale(rows_b, j + 1)
                pltpu.sync_copy(
                    rows_b, agg_sh.at[dst1.at[pl.ds((j + 1) * C, C)]],
                    add=True)
                return carry

            lax.fori_loop(0, NCHUNK // 2, passb, 0)
            jt = NCHUNK - 1
            pltpu.make_async_copy(
                feat_h.at[src1.at[pl.ds(jt * C, C)]], rows_a, gsem_a).wait()
            _scale(rows_a, jt)
            pltpu.sync_copy(rows_a,
                            agg_sh.at[dst1.at[pl.ds(jt * C, C)]], add=True)

            plsc.subcore_barrier()

            pltpu.sync_copy(agg_sh.at[pl.ds(base, RPT)],
                            agg_outs[p].at[cid, pl.ds(base, RPT)])
            if p + 1 < nparts:
                pltpu.sync_copy(z_h, agg_sh.at[pl.ds(base, RPT)])

    return sc_edge


_tc_feat1 = _make_tc_feat(H1_F)
_tc_mid = _make_tc_mid(H1_F, OUT_F)
_tc_final = _make_tc_final(OUT_F)
_sc_edge1 = _make_sc_edge(2)
_sc_edge2 = _make_sc_edge(1)


def kernel(x, edge_index, W1, attn_l1, attn_r1, b1, W2, attn_l2, attn_r2, b2):
    src = edge_index[0]
    dst = edge_index[1]
    x_pad = jnp.pad(x, ((0, N_PAD - N), (0, 0)))
    z = jnp.zeros((RPT, FH), jnp.float32)

    f1a, f1b, el3, er3, m1 = _tc_feat1(x_pad, W1, attn_l1, attn_r1)
    m16 = jnp.full((L,), m1[0, 0] + m1[0, 1], jnp.float32)
    aggA, aggB, den1 = _sc_edge1(src, dst,
                                 el3.reshape(N_PAD), er3.reshape(N_PAD), m16,
                                 f1a, f1b, z)

    feat2, el3b, er3b, m2 = _tc_mid(aggA, aggB, den1.reshape(NW, N_PAD), b1,
                                    W2, attn_l2, attn_r2)
    m16b = jnp.full((L,), m2[0, 0] + m2[0, 1], jnp.float32)
    agg2, den2 = _sc_edge2(src, dst,
                           el3b.reshape(N_PAD), er3b.reshape(N_PAD), m16b,
                           feat2, z)

    return _tc_final(agg2, den2.reshape(NW, N_PAD), b2)


# speedup vs baseline: 3.4509x; 3.4509x over previous
import functools

import jax
import jax.numpy as jnp
from jax import lax
from jax.experimental import pallas as pl
from jax.experimental.pallas import tpu as pltpu
from jax.experimental.pallas import tpu_sc as plsc

N = 10000
E = 320000
IN_F = 128
H1_F = 128
OUT_F = 64
FQ = 32

NC = 2
NS = 16
L = 16
NW = NC * NS
EPW = E // NW
C = 80
NCHUNK = EPW // C

BLK = 1024
NB = 10
N_PAD = NB * BLK
N2 = 2 * N_PAD
RPT2 = N2 // NS
DROW2 = N2 // L

_NEG_INF = -3.0e38


def _attn_tail(i, feat, al_ref, ar_ref, el_ref, er_ref, m_ref):
    el = jnp.sum(feat * al_ref[...][None, :], axis=1)
    er = jnp.sum(feat * ar_ref[...][None, :], axis=1)
    el_ref[0, 0, :] = el
    er_ref[0, 0, :] = er

    @pl.when(i == 0)
    def _():
        m_ref[0, 0] = _NEG_INF
        m_ref[0, 1] = _NEG_INF

    m_ref[0, 0] = jnp.maximum(m_ref[0, 0], jnp.max(el))
    m_ref[0, 1] = jnp.maximum(m_ref[0, 1], jnp.max(er))


def _feat_body(h_ref, w_ref, al_ref, ar_ref, feat_ref, el_ref, er_ref, m_ref):
    i = pl.program_id(0)
    feat = jnp.dot(h_ref[...], w_ref[...], preferred_element_type=jnp.float32)
    feat_ref[...] = feat
    _attn_tail(i, feat, al_ref, ar_ref, el_ref, er_ref, m_ref)


def _make_tc_feat(F_in, F_out):
    return pl.pallas_call(
        _feat_body,
        grid=(NB,),
        in_specs=[
            pl.BlockSpec((BLK, F_in), lambda i: (i, 0)),
            pl.BlockSpec((F_in, F_out), lambda i: (0, 0)),
            pl.BlockSpec((F_out,), lambda i: (0,)),
            pl.BlockSpec((F_out,), lambda i: (0,)),
        ],
        out_specs=[
            pl.BlockSpec((BLK, F_out), lambda i: (i, 0)),
            pl.BlockSpec((1, 1, BLK), lambda i: (i, 0, 0)),
            pl.BlockSpec((1, 1, BLK), lambda i: (i, 0, 0)),
            pl.BlockSpec((1, 2), lambda i: (0, 0), memory_space=pltpu.SMEM),
        ],
        out_shape=[
            jax.ShapeDtypeStruct((N_PAD, F_out), jnp.float32),
            jax.ShapeDtypeStruct((NB, 1, BLK), jnp.float32),
            jax.ShapeDtypeStruct((NB, 1, BLK), jnp.float32),
            jax.ShapeDtypeStruct((1, 2), jnp.float32),
        ],
    )


def _prep_body(feat_ref, el_ref, er_ref, m_ref, *out_refs):
    nq = len(out_refs) - 2
    g_refs = out_refs[:nq]
    p_ref, q_ref = out_refs[nq:]
    max_el = m_ref[0, 0]
    max_er = m_ref[0, 1]
    mt = max_el + max_er
    el = el_ref[0, 0, :]
    er = er_ref[0, 0, :]
    P = jnp.exp(el - max_el)
    P2 = jnp.exp(0.2 * (el - max_el))
    Q = jnp.exp(er - max_er)
    Q2 = jnp.exp(0.2 * (er - max_er) - 0.8 * mt)
    p_ref[0, 0, 0, :] = P
    p_ref[1, 0, 0, :] = P2
    q_ref[0, 0, 0, :] = Q
    q_ref[1, 0, 0, :] = Q2
    feat = feat_ref[...]
    for q in range(nq):
        cols = feat[:, q * FQ:(q + 1) * FQ]
        g_refs[q][0] = cols * P[:, None]
        g_refs[q][1] = cols * P2[:, None]


def _make_tc_prep(F):
    nq = F // FQ
    return pl.pallas_call(
        _prep_body,
        grid=(NB,),
        in_specs=[
            pl.BlockSpec((BLK, F), lambda i: (i, 0)),
            pl.BlockSpec((1, 1, BLK), lambda i: (i, 0, 0)),
            pl.BlockSpec((1, 1, BLK), lambda i: (i, 0, 0)),
            pl.BlockSpec((1, 2), lambda i: (0, 0), memory_space=pltpu.SMEM),
        ],
        out_specs=(
            [pl.BlockSpec((2, BLK, FQ), lambda i: (0, i, 0))] * nq
            + [pl.BlockSpec((2, 1, 1, BLK), lambda i: (0, i, 0, 0))] * 2
        ),
        out_shape=(
            [jax.ShapeDtypeStruct((2, N_PAD, FQ), jnp.float32)] * nq
            + [jax.ShapeDtypeStruct((2, NB, 1, BLK), jnp.float32)] * 2
        ),
    )


def _recombine(acc_refs, den_ref, q_ref, b_ref):
    q = q_ref[0, 0, 0, :]
    q2 = q_ref[1, 0, 0, :]
    den = (q * jnp.sum(den_ref[:, 0, :], axis=0)
           + q2 * jnp.sum(den_ref[:, 1, :], axis=0))
    inv = 1.0 / (den + 1e-16)
    parts = []
    for a in acc_refs:
        pos = a[0, 0] + a[1, 0]
        neg = a[0, 1] + a[1, 1]
        parts.append(q[:, None] * pos + q2[:, None] * neg)
    h = jnp.concatenate(parts, axis=1) * inv[:, None] + b_ref[...][None, :]
    return jnp.where(h >= 0.0, h, 0.01 * h)


def _mid_body(a0, a1, a2, a3, den_ref, q_ref, b_ref, w_ref, al_ref, ar_ref,
              feat_ref, el_ref, er_ref, m_ref):
    i = pl.program_id(0)
    h = _recombine((a0, a1, a2, a3), den_ref, q_ref, b_ref)
    feat = jnp.dot(h, w_ref[...], preferred_element_type=jnp.float32)
    feat_ref[...] = feat
    _attn_tail(i, feat, al_ref, ar_ref, el_ref, er_ref, m_ref)


def _make_tc_mid(F_in, F_out):
    nq = F_in // FQ
    return pl.pallas_call(
        _mid_body,
        grid=(NB,),
        in_specs=(
            [pl.BlockSpec((NC, 2, BLK, FQ), lambda i: (0, 0, i, 0))] * nq
            + [
                pl.BlockSpec((NW, 2, BLK), lambda i: (0, 0, i)),
                pl.BlockSpec((2, 1, 1, BLK), lambda i: (0, i, 0, 0)),
                pl.BlockSpec((F_in,), lambda i: (0,)),
                pl.BlockSpec((F_in, F_out), lambda i: (0, 0)),
                pl.BlockSpec((F_out,), lambda i: (0,)),
                pl.BlockSpec((F_out,), lambda i: (0,)),
            ]
        ),
        out_specs=[
            pl.BlockSpec((BLK, F_out), lambda i: (i, 0)),
            pl.BlockSpec((1, 1, BLK), lambda i: (i, 0, 0)),
            pl.BlockSpec((1, 1, BLK), lambda i: (i, 0, 0)),
            pl.BlockSpec((1, 2), lambda i: (0, 0), memory_space=pltpu.SMEM),
        ],
        out_shape=[
            jax.ShapeDtypeStruct((N_PAD, F_out), jnp.float32),
            jax.ShapeDtypeStruct((NB, 1, BLK), jnp.float32),
            jax.ShapeDtypeStruct((NB, 1, BLK), jnp.float32),
            jax.ShapeDtypeStruct((1, 2), jnp.float32),
        ],
    )


def _final_body(a0, a1, den_ref, q_ref, b_ref, out_ref):
    i = pl.program_id(0)
    h = _recombine((a0, a1), den_ref, q_ref, b_ref)
    row = i * BLK + lax.broadcasted_iota(jnp.int32, (BLK, 1), 0)
    h = jnp.where(row < N, h, 0.0)

    @pl.when(i == 0)
    def _():
        out_ref[...] = jnp.zeros_like(out_ref)

    out_ref[...] += jnp.sum(h, axis=0, keepdims=True)

    @pl.when(i == NB - 1)
    def _():
        out_ref[...] *= jnp.float32(1.0 / N)


def _make_tc_final(F):
    nq = F // FQ
    return pl.pallas_call(
        _final_body,
        grid=(NB,),
        in_specs=(
            [pl.BlockSpec((NC, 2, BLK, FQ), lambda i: (0, 0, i, 0))] * nq
            + [
                pl.BlockSpec((NW, 2, BLK), lambda i: (0, 0, i)),
                pl.BlockSpec((2, 1, 1, BLK), lambda i: (0, i, 0, 0)),
                pl.BlockSpec((F,), lambda i: (0,)),
            ]
        ),
        out_specs=pl.BlockSpec((1, F), lambda i: (0, 0)),
        out_shape=jax.ShapeDtypeStruct((1, F), jnp.float32),
    )


def _make_sc_edge(nparts):
    mesh = plsc.VectorSubcoreMesh(core_axis_name="c", subcore_axis_name="s")

    @functools.partial(
        pl.kernel,
        out_type=(
            [jax.ShapeDtypeStruct((NC, N2, FQ), jnp.float32)] * nparts
            + [jax.ShapeDtypeStruct((NW, DROW2, L), jnp.float32)]
        ),
        mesh=mesh,
        compiler_params=pltpu.CompilerParams(
            needs_layout_passes=False, use_tc_tiling_on_sc=False),
        scratch_types=[
            pltpu.VMEM((EPW,), jnp.int32),
            pltpu.VMEM((EPW,), jnp.int32),
            pltpu.VMEM((N_PAD,), jnp.float32),
            pltpu.VMEM((N_PAD,), jnp.float32),
            pltpu.VMEM((N2,), jnp.float32),
            pltpu.VMEM((DROW2, L), jnp.float32),
            pltpu.VMEM((C, FQ), jnp.float32),
            pltpu.VMEM((C, FQ), jnp.float32),
            pltpu.SemaphoreType.DMA,
            pltpu.SemaphoreType.DMA,
            pltpu.VMEM_SHARED((N2, FQ), jnp.float32),
        ],
    )
    def sc_edge(*args):
        (src1_h, dst1_h, el_h, er_h, pc_h) = args[:5]
        g_hs = args[5:5 + nparts]
        z_h = args[5 + nparts]
        agg_outs = args[6 + nparts:6 + 2 * nparts]
        den_out = args[6 + 2 * nparts]
        (src1, dst1, el_v, er_v, pc_v, den_v,
         rows_a, rows_b, gsem_a, gsem_b, agg_sh) = args[7 + 2 * nparts:]

        cid = lax.axis_index("c")
        sid = lax.axis_index("s")
        wid = cid * NS + sid
        base = sid * RPT2

        pltpu.sync_copy(src1_h.at[pl.ds(wid * EPW, EPW)], src1)
        pltpu.sync_copy(dst1_h.at[pl.ds(wid * EPW, EPW)], dst1)
        pltpu.sync_copy(el_h, el_v)
        pltpu.sync_copy(er_h, er_v)
        pltpu.sync_copy(pc_h, pc_v)

        pltpu.sync_copy(z_h, agg_sh.at[pl.ds(base, RPT2)])

        zvec = jnp.zeros((L,), jnp.float32)

        def zden(r, carry):
            den_v[r, pl.ds(0, L)] = zvec
            return carry

        lax.fori_loop(0, DROW2, zden, 0)

        def passa(t, carry):
            s_idx = src1[pl.ds(t * L, L)]
            d_idx = dst1[pl.ds(t * L, L)]
            e = plsc.load_gather(el_v, [s_idx]) + plsc.load_gather(er_v, [d_idx])
            adj = jnp.where(e < 0.0, N_PAD, 0).astype(jnp.int32)
            si = s_idx + adj
            di = d_idx + adj
            src1[pl.ds(t * L, L)] = si
            dst1[pl.ds(t * L, L)] = di
            v = plsc.load_gather(pc_v, [si])
            plsc.addupdate_scatter(
                den_v, [lax.shift_right_logical(di, 4),
                        lax.bitwise_and(di, 15)], v)
            return carry

        lax.fori_loop(0, EPW // L, passa, 0)
        pltpu.sync_copy(den_v, den_out.at[wid])

        for p in range(nparts):
            plsc.subcore_barrier()

            g_h = g_hs[p]
            pltpu.async_copy(g_h.at[src1.at[pl.ds(0, C)]], rows_a, gsem_a)

            def passb(i, carry):
                j = 2 * i
                pltpu.async_copy(
                    g_h.at[src1.at[pl.ds((j + 1) * C, C)]], rows_b, gsem_b)
                pltpu.make_async_copy(
                    g_h.at[src1.at[pl.ds(j * C, C)]], rows_a, gsem_a).wait()
                pltpu.sync_copy(rows_a,
                                agg_sh.at[dst1.at[pl.ds(j * C, C)]], add=True)

                @pl.when(j + 2 < NCHUNK)
                def _():
                    pltpu.async_copy(
                        g_h.at[src1.at[pl.ds((j + 2) * C, C)]],
                        rows_a, gsem_a)

                pltpu.make_async_copy(
                    g_h.at[src1.at[pl.ds((j + 1) * C, C)]],
                    rows_b, gsem_b).wait()
                pltpu.sync_copy(
                    rows_b, agg_sh.at[dst1.at[pl.ds((j + 1) * C, C)]],
                    add=True)
                return carry

            lax.fori_loop(0, NCHUNK // 2, passb, 0)
            jt = NCHUNK - 1
            pltpu.make_async_copy(
                g_h.at[src1.at[pl.ds(jt * C, C)]], rows_a, gsem_a).wait()
            pltpu.sync_copy(rows_a,
                            agg_sh.at[dst1.at[pl.ds(jt * C, C)]], add=True)

            plsc.subcore_barrier()

            pltpu.sync_copy(agg_sh.at[pl.ds(base, RPT2)],
                            agg_outs[p].at[cid, pl.ds(base, RPT2)])
            if p + 1 < nparts:
                pltpu.sync_copy(z_h, agg_sh.at[pl.ds(base, RPT2)])

    return sc_edge


_tc_feat1 = _make_tc_feat(IN_F, H1_F)
_tc_prep1 = _make_tc_prep(H1_F)
_tc_mid = _make_tc_mid(H1_F, OUT_F)
_tc_prep2 = _make_tc_prep(OUT_F)
_tc_final = _make_tc_final(OUT_F)
_sc_edge1 = _make_sc_edge(H1_F // FQ)
_sc_edge2 = _make_sc_edge(OUT_F // FQ)


def kernel(x, edge_index, W1, attn_l1, attn_r1, b1, W2, attn_l2, attn_r2, b2):
    src = edge_index[0]
    dst = edge_index[1]
    x_pad = jnp.pad(x, ((0, N_PAD - N), (0, 0)))
    z = jnp.zeros((RPT2, FQ), jnp.float32)

    feat1, el3, er3, m1 = _tc_feat1(x_pad, W1, attn_l1, attn_r1)
    g10, g11, g12, g13, p1, q1 = _tc_prep1(feat1, el3, er3, m1)
    a0, a1, a2, a3, den1 = _sc_edge1(
        src, dst, el3.reshape(N_PAD), er3.reshape(N_PAD), p1.reshape(N2),
        g10.reshape(N2, FQ), g11.reshape(N2, FQ),
        g12.reshape(N2, FQ), g13.reshape(N2, FQ), z)

    feat2, el3b, er3b, m2 = _tc_mid(
        a0.reshape(NC, 2, N_PAD, FQ), a1.reshape(NC, 2, N_PAD, FQ),
        a2.reshape(NC, 2, N_PAD, FQ), a3.reshape(NC, 2, N_PAD, FQ),
        den1.reshape(NW, 2, N_PAD), q1, b1, W2, attn_l2, attn_r2)
    g20, g21, p2, q2 = _tc_prep2(feat2, el3b, er3b, m2)
    c0, c1, den2 = _sc_edge2(
        src, dst, el3b.reshape(N_PAD), er3b.reshape(N_PAD), p2.reshape(N2),
        g20.reshape(N2, FQ), g21.reshape(N2, FQ), z)

    return _tc_final(c0.reshape(NC, 2, N_PAD, FQ), c1.reshape(NC, 2, N_PAD, FQ),
                     den2.reshape(NW, 2, N_PAD), q2, b2)


# speedup vs baseline: 4.8882x; 1.4165x over previous
import functools

import jax
import jax.numpy as jnp
from jax import lax
from jax.experimental import pallas as pl
from jax.experimental.pallas import tpu as pltpu
from jax.experimental.pallas import tpu_sc as plsc

N = 10000
E = 320000
IN_F = 128
H1_F = 128
OUT_F = 64
FQ = 64

NC = 2
NS = 16
L = 16
NW = NC * NS
EPW = E // NW
C = 80
NCHUNK = EPW // C

BLK = 1024
NB = 10
N_PAD = NB * BLK
N2 = 2 * N_PAD
RPT2 = N2 // NS
DROW2 = N2 // L

_NEG_INF = -3.0e38


def _attn_tail(i, feat, al_ref, ar_ref, el_ref, er_ref, m_ref):
    el = jnp.sum(feat * al_ref[...][None, :], axis=1)
    er = jnp.sum(feat * ar_ref[...][None, :], axis=1)
    el_ref[0, 0, :] = el
    er_ref[0, 0, :] = er

    @pl.when(i == 0)
    def _():
        m_ref[0, 0] = _NEG_INF
        m_ref[0, 1] = _NEG_INF

    m_ref[0, 0] = jnp.maximum(m_ref[0, 0], jnp.max(el))
    m_ref[0, 1] = jnp.maximum(m_ref[0, 1], jnp.max(er))


def _feat_body(h_ref, w_ref, al_ref, ar_ref, feat_ref, el_ref, er_ref, m_ref):
    i = pl.program_id(0)
    feat = jnp.dot(h_ref[...], w_ref[...], preferred_element_type=jnp.float32)
    feat_ref[...] = feat
    _attn_tail(i, feat, al_ref, ar_ref, el_ref, er_ref, m_ref)


def _make_tc_feat(F_in, F_out):
    return pl.pallas_call(
        _feat_body,
        grid=(NB,),
        in_specs=[
            pl.BlockSpec((BLK, F_in), lambda i: (i, 0)),
            pl.BlockSpec((F_in, F_out), lambda i: (0, 0)),
            pl.BlockSpec((F_out,), lambda i: (0,)),
            pl.BlockSpec((F_out,), lambda i: (0,)),
        ],
        out_specs=[
            pl.BlockSpec((BLK, F_out), lambda i: (i, 0)),
            pl.BlockSpec((1, 1, BLK), lambda i: (i, 0, 0)),
            pl.BlockSpec((1, 1, BLK), lambda i: (i, 0, 0)),
            pl.BlockSpec((1, 2), lambda i: (0, 0), memory_space=pltpu.SMEM),
        ],
        out_shape=[
            jax.ShapeDtypeStruct((N_PAD, F_out), jnp.float32),
            jax.ShapeDtypeStruct((NB, 1, BLK), jnp.float32),
            jax.ShapeDtypeStruct((NB, 1, BLK), jnp.float32),
            jax.ShapeDtypeStruct((1, 2), jnp.float32),
        ],
    )


def _prep_body(feat_ref, el_ref, er_ref, m_ref, *out_refs):
    nq = len(out_refs) - 2
    g_refs = out_refs[:nq]
    p_ref, q_ref = out_refs[nq:]
    max_el = m_ref[0, 0]
    max_er = m_ref[0, 1]
    mt = max_el + max_er
    el = el_ref[0, 0, :]
    er = er_ref[0, 0, :]
    P = jnp.exp(el - max_el)
    P2 = jnp.exp(0.2 * (el - max_el))
    Q = jnp.exp(er - max_er)
    Q2 = jnp.exp(0.2 * (er - max_er) - 0.8 * mt)
    p_ref[0, 0, 0, :] = P
    p_ref[1, 0, 0, :] = P2
    q_ref[0, 0, 0, :] = Q
    q_ref[1, 0, 0, :] = Q2
    feat = feat_ref[...]
    for q in range(nq):
        cols = feat[:, q * FQ:(q + 1) * FQ]
        g_refs[q][0] = (cols * P[:, None]).astype(jnp.bfloat16)
        g_refs[q][1] = (cols * P2[:, None]).astype(jnp.bfloat16)


def _make_tc_prep(F):
    nq = F // FQ
    return pl.pallas_call(
        _prep_body,
        grid=(NB,),
        in_specs=[
            pl.BlockSpec((BLK, F), lambda i: (i, 0)),
            pl.BlockSpec((1, 1, BLK), lambda i: (i, 0, 0)),
            pl.BlockSpec((1, 1, BLK), lambda i: (i, 0, 0)),
            pl.BlockSpec((1, 2), lambda i: (0, 0), memory_space=pltpu.SMEM),
        ],
        out_specs=(
            [pl.BlockSpec((2, BLK, FQ), lambda i: (0, i, 0))] * nq
            + [pl.BlockSpec((2, 1, 1, BLK), lambda i: (0, i, 0, 0))] * 2
        ),
        out_shape=(
            [jax.ShapeDtypeStruct((2, N_PAD, FQ), jnp.bfloat16)] * nq
            + [jax.ShapeDtypeStruct((2, NB, 1, BLK), jnp.float32)] * 2
        ),
    )


def _recombine(acc_refs, den_ref, q_ref, b_ref):
    q = q_ref[0, 0, 0, :]
    q2 = q_ref[1, 0, 0, :]
    den = (q * jnp.sum(den_ref[:, 0, :], axis=0)
           + q2 * jnp.sum(den_ref[:, 1, :], axis=0))
    inv = 1.0 / (den + 1e-16)
    parts = []
    for a in acc_refs:
        af = a[...].astype(jnp.float32)
        pos = af[0, 0] + af[1, 0]
        neg = af[0, 1] + af[1, 1]
        parts.append(q[:, None] * pos + q2[:, None] * neg)
    h = jnp.concatenate(parts, axis=1) * inv[:, None] + b_ref[...][None, :]
    return jnp.where(h >= 0.0, h, 0.01 * h)


def _mid_body(a0, a1, den_ref, q_ref, b_ref, w_ref, al_ref, ar_ref,
              feat_ref, el_ref, er_ref, m_ref):
    i = pl.program_id(0)
    h = _recombine((a0, a1), den_ref, q_ref, b_ref)
    feat = jnp.dot(h, w_ref[...], preferred_element_type=jnp.float32)
    feat_ref[...] = feat
    _attn_tail(i, feat, al_ref, ar_ref, el_ref, er_ref, m_ref)


def _make_tc_mid(F_in, F_out):
    nq = F_in // FQ
    return pl.pallas_call(
        _mid_body,
        grid=(NB,),
        in_specs=(
            [pl.BlockSpec((NC, 2, BLK, FQ), lambda i: (0, 0, i, 0))] * nq
            + [
                pl.BlockSpec((NW, 2, BLK), lambda i: (0, 0, i)),
                pl.BlockSpec((2, 1, 1, BLK), lambda i: (0, i, 0, 0)),
                pl.BlockSpec((F_in,), lambda i: (0,)),
                pl.BlockSpec((F_in, F_out), lambda i: (0, 0)),
                pl.BlockSpec((F_out,), lambda i: (0,)),
                pl.BlockSpec((F_out,), lambda i: (0,)),
            ]
        ),
        out_specs=[
            pl.BlockSpec((BLK, F_out), lambda i: (i, 0)),
            pl.BlockSpec((1, 1, BLK), lambda i: (i, 0, 0)),
            pl.BlockSpec((1, 1, BLK), lambda i: (i, 0, 0)),
            pl.BlockSpec((1, 2), lambda i: (0, 0), memory_space=pltpu.SMEM),
        ],
        out_shape=[
            jax.ShapeDtypeStruct((N_PAD, F_out), jnp.float32),
            jax.ShapeDtypeStruct((NB, 1, BLK), jnp.float32),
            jax.ShapeDtypeStruct((NB, 1, BLK), jnp.float32),
            jax.ShapeDtypeStruct((1, 2), jnp.float32),
        ],
    )


def _final_body(a0, den_ref, q_ref, b_ref, out_ref):
    i = pl.program_id(0)
    h = _recombine((a0,), den_ref, q_ref, b_ref)
    row = i * BLK + lax.broadcasted_iota(jnp.int32, (BLK, 1), 0)
    h = jnp.where(row < N, h, 0.0)

    @pl.when(i == 0)
    def _():
        out_ref[...] = jnp.zeros_like(out_ref)

    out_ref[...] += jnp.sum(h, axis=0, keepdims=True)

    @pl.when(i == NB - 1)
    def _():
        out_ref[...] *= jnp.float32(1.0 / N)


def _make_tc_final(F):
    nq = F // FQ
    return pl.pallas_call(
        _final_body,
        grid=(NB,),
        in_specs=(
            [pl.BlockSpec((NC, 2, BLK, FQ), lambda i: (0, 0, i, 0))] * nq
            + [
                pl.BlockSpec((NW, 2, BLK), lambda i: (0, 0, i)),
                pl.BlockSpec((2, 1, 1, BLK), lambda i: (0, i, 0, 0)),
                pl.BlockSpec((F,), lambda i: (0,)),
            ]
        ),
        out_specs=pl.BlockSpec((1, F), lambda i: (0, 0)),
        out_shape=jax.ShapeDtypeStruct((1, F), jnp.float32),
    )


def _make_sc_edge(nparts):
    mesh = plsc.VectorSubcoreMesh(core_axis_name="c", subcore_axis_name="s")

    @functools.partial(
        pl.kernel,
        out_type=(
            [jax.ShapeDtypeStruct((NC, N2, FQ), jnp.bfloat16)] * nparts
            + [jax.ShapeDtypeStruct((NW, DROW2, L), jnp.float32)]
        ),
        mesh=mesh,
        compiler_params=pltpu.CompilerParams(
            needs_layout_passes=False, use_tc_tiling_on_sc=False),
        scratch_types=[
            pltpu.VMEM((EPW,), jnp.int32),
            pltpu.VMEM((EPW,), jnp.int32),
            pltpu.VMEM((N_PAD,), jnp.float32),
            pltpu.VMEM((N_PAD,), jnp.float32),
            pltpu.VMEM((N2,), jnp.float32),
            pltpu.VMEM((DROW2, L), jnp.float32),
            pltpu.VMEM((C, FQ), jnp.bfloat16),
            pltpu.VMEM((C, FQ), jnp.bfloat16),
            pltpu.SemaphoreType.DMA,
            pltpu.SemaphoreType.DMA,
            pltpu.VMEM_SHARED((N2, FQ), jnp.bfloat16),
        ],
    )
    def sc_edge(*args):
        (src1_h, dst1_h, el_h, er_h, pc_h) = args[:5]
        g_hs = args[5:5 + nparts]
        z_h = args[5 + nparts]
        agg_outs = args[6 + nparts:6 + 2 * nparts]
        den_out = args[6 + 2 * nparts]
        (src1, dst1, el_v, er_v, pc_v, den_v,
         rows_a, rows_b, gsem_a, gsem_b, agg_sh) = args[7 + 2 * nparts:]

        cid = lax.axis_index("c")
        sid = lax.axis_index("s")
        wid = cid * NS + sid
        base = sid * RPT2

        pltpu.sync_copy(src1_h.at[pl.ds(wid * EPW, EPW)], src1)
        pltpu.sync_copy(dst1_h.at[pl.ds(wid * EPW, EPW)], dst1)
        pltpu.sync_copy(el_h, el_v)
        pltpu.sync_copy(er_h, er_v)
        pltpu.sync_copy(pc_h, pc_v)

        pltpu.sync_copy(z_h, agg_sh.at[pl.ds(base, RPT2)])

        zvec = jnp.zeros((L,), jnp.float32)

        def zden(r, carry):
            den_v[r, pl.ds(0, L)] = zvec
            return carry

        lax.fori_loop(0, DROW2, zden, 0)

        def passa(t, carry):
            s_idx = src1[pl.ds(t * L, L)]
            d_idx = dst1[pl.ds(t * L, L)]
            e = plsc.load_gather(el_v, [s_idx]) + plsc.load_gather(er_v, [d_idx])
            adj = jnp.where(e < 0.0, N_PAD, 0).astype(jnp.int32)
            si = s_idx + adj
            di = d_idx + adj
            src1[pl.ds(t * L, L)] = si
            dst1[pl.ds(t * L, L)] = di
            v = plsc.load_gather(pc_v, [si])
            plsc.addupdate_scatter(
                den_v, [lax.shift_right_logical(di, 4),
                        lax.bitwise_and(di, 15)], v)
            return carry

        lax.fori_loop(0, EPW // L, passa, 0)
        pltpu.sync_copy(den_v, den_out.at[wid])

        for p in range(nparts):
            plsc.subcore_barrier()

            g_h = g_hs[p]
            pltpu.async_copy(g_h.at[src1.at[pl.ds(0, C)]], rows_a, gsem_a)

            def passb(i, carry):
                j = 2 * i
                pltpu.async_copy(
                    g_h.at[src1.at[pl.ds((j + 1) * C, C)]], rows_b, gsem_b)
                pltpu.make_async_copy(
                    g_h.at[src1.at[pl.ds(j * C, C)]], rows_a, gsem_a).wait()
                pltpu.sync_copy(rows_a,
                                agg_sh.at[dst1.at[pl.ds(j * C, C)]], add=True)

                @pl.when(j + 2 < NCHUNK)
                def _():
                    pltpu.async_copy(
                        g_h.at[src1.at[pl.ds((j + 2) * C, C)]],
                        rows_a, gsem_a)

                pltpu.make_async_copy(
                    g_h.at[src1.at[pl.ds((j + 1) * C, C)]],
                    rows_b, gsem_b).wait()
                pltpu.sync_copy(
                    rows_b, agg_sh.at[dst1.at[pl.ds((j + 1) * C, C)]],
                    add=True)
                return carry

            lax.fori_loop(0, NCHUNK // 2, passb, 0)
            jt = NCHUNK - 1
            pltpu.make_async_copy(
                g_h.at[src1.at[pl.ds(jt * C, C)]], rows_a, gsem_a).wait()
            pltpu.sync_copy(rows_a,
                            agg_sh.at[dst1.at[pl.ds(jt * C, C)]], add=True)

            plsc.subcore_barrier()

            pltpu.sync_copy(agg_sh.at[pl.ds(base, RPT2)],
                            agg_outs[p].at[cid, pl.ds(base, RPT2)])
            if p + 1 < nparts:
                pltpu.sync_copy(z_h, agg_sh.at[pl.ds(base, RPT2)])

    return sc_edge


_tc_feat1 = _make_tc_feat(IN_F, H1_F)
_tc_prep1 = _make_tc_prep(H1_F)
_tc_mid = _make_tc_mid(H1_F, OUT_F)
_tc_prep2 = _make_tc_prep(OUT_F)
_tc_final = _make_tc_final(OUT_F)
_sc_edge1 = _make_sc_edge(H1_F // FQ)
_sc_edge2 = _make_sc_edge(OUT_F // FQ)


def kernel(x, edge_index, W1, attn_l1, attn_r1, b1, W2, attn_l2, attn_r2, b2):
    src = edge_index[0]
    dst = edge_index[1]
    x_pad = jnp.pad(x, ((0, N_PAD - N), (0, 0)))
    z = jnp.zeros((RPT2, FQ), jnp.bfloat16)

    feat1, el3, er3, m1 = _tc_feat1(x_pad, W1, attn_l1, attn_r1)
    g10, g11, p1, q1 = _tc_prep1(feat1, el3, er3, m1)
    a0, a1, den1 = _sc_edge1(
        src, dst, el3.reshape(N_PAD), er3.reshape(N_PAD), p1.reshape(N2),
        g10.reshape(N2, FQ), g11.reshape(N2, FQ), z)

    feat2, el3b, er3b, m2 = _tc_mid(
        a0.reshape(NC, 2, N_PAD, FQ), a1.reshape(NC, 2, N_PAD, FQ),
        den1.reshape(NW, 2, N_PAD), q1, b1, W2, attn_l2, attn_r2)
    g20, p2, q2 = _tc_prep2(feat2, el3b, er3b, m2)
    c0, den2 = _sc_edge2(
        src, dst, el3b.reshape(N_PAD), er3b.reshape(N_PAD), p2.reshape(N2),
        g20.reshape(N2, FQ), z)

    return _tc_final(c0.reshape(NC, 2, N_PAD, FQ),
                     den2.reshape(NW, 2, N_PAD), q2, b2)


# speedup vs baseline: 5.4914x; 1.1234x over previous
import functools

import jax
import jax.numpy as jnp
from jax import lax
from jax.experimental import pallas as pl
from jax.experimental.pallas import tpu as pltpu
from jax.experimental.pallas import tpu_sc as plsc

N = 10000
E = 320000
IN_F = 128
H1_F = 128
OUT_F = 64
FQ = 64

NC = 2
NS = 16
L = 16
NW = NC * NS
EPW = E // NW
C = 80
NCHUNK = EPW // C

BLK = 1024
NB = 10
N_PAD = NB * BLK
N2 = 2 * N_PAD
RPT2 = N2 // NS
DROW2 = N2 // L

_NEG_INF = -3.0e38


def _attn_tail(i, feat, al_ref, ar_ref, el_ref, er_ref, m_ref):
    el = jnp.sum(feat * al_ref[...][None, :], axis=1)
    er = jnp.sum(feat * ar_ref[...][None, :], axis=1)
    el_ref[0, 0, :] = el
    er_ref[0, 0, :] = er

    @pl.when(i == 0)
    def _():
        m_ref[0, 0] = _NEG_INF
        m_ref[0, 1] = _NEG_INF

    m_ref[0, 0] = jnp.maximum(m_ref[0, 0], jnp.max(el))
    m_ref[0, 1] = jnp.maximum(m_ref[0, 1], jnp.max(er))


def _feat_body(h_ref, w_ref, al_ref, ar_ref, feat_ref, el_ref, er_ref, m_ref):
    i = pl.program_id(0)
    feat = jnp.dot(h_ref[...], w_ref[...], preferred_element_type=jnp.float32)
    feat_ref[...] = feat
    _attn_tail(i, feat, al_ref, ar_ref, el_ref, er_ref, m_ref)


def _make_tc_feat(F_in, F_out):
    return pl.pallas_call(
        _feat_body,
        grid=(NB,),
        in_specs=[
            pl.BlockSpec((BLK, F_in), lambda i: (i, 0)),
            pl.BlockSpec((F_in, F_out), lambda i: (0, 0)),
            pl.BlockSpec((F_out,), lambda i: (0,)),
            pl.BlockSpec((F_out,), lambda i: (0,)),
        ],
        out_specs=[
            pl.BlockSpec((BLK, F_out), lambda i: (i, 0)),
            pl.BlockSpec((1, 1, BLK), lambda i: (i, 0, 0)),
            pl.BlockSpec((1, 1, BLK), lambda i: (i, 0, 0)),
            pl.BlockSpec((1, 2), lambda i: (0, 0), memory_space=pltpu.SMEM),
        ],
        out_shape=[
            jax.ShapeDtypeStruct((N_PAD, F_out), jnp.float32),
            jax.ShapeDtypeStruct((NB, 1, BLK), jnp.float32),
            jax.ShapeDtypeStruct((NB, 1, BLK), jnp.float32),
            jax.ShapeDtypeStruct((1, 2), jnp.float32),
        ],
    )


def _prep_body(feat_ref, el_ref, er_ref, m_ref, *out_refs):
    nq = len(out_refs) - 1
    g_refs = out_refs[:nq]
    q_ref = out_refs[nq]
    max_el = m_ref[0, 0]
    max_er = m_ref[0, 1]
    mt = max_el + max_er
    el = el_ref[0, 0, :]
    er = er_ref[0, 0, :]
    P = jnp.exp(el - max_el)
    P2 = jnp.exp(0.2 * (el - max_el))
    Q = jnp.exp(er - max_er)
    Q2 = jnp.exp(0.2 * (er - max_er) - 0.8 * mt)
    q_ref[0, 0, 0, :] = Q
    q_ref[1, 0, 0, :] = Q2
    feat = feat_ref[...]
    for q in range(nq):
        cols = feat[:, q * FQ:(q + 1) * FQ]
        g_refs[q][0] = (cols * P[:, None]).astype(jnp.bfloat16)
        g_refs[q][1] = (cols * P2[:, None]).astype(jnp.bfloat16)


def _make_tc_prep(F):
    nq = F // FQ
    return pl.pallas_call(
        _prep_body,
        grid=(NB,),
        in_specs=[
            pl.BlockSpec((BLK, F), lambda i: (i, 0)),
            pl.BlockSpec((1, 1, BLK), lambda i: (i, 0, 0)),
            pl.BlockSpec((1, 1, BLK), lambda i: (i, 0, 0)),
            pl.BlockSpec((1, 2), lambda i: (0, 0), memory_space=pltpu.SMEM),
        ],
        out_specs=(
            [pl.BlockSpec((2, BLK, FQ), lambda i: (0, i, 0))] * nq
            + [pl.BlockSpec((2, 1, 1, BLK), lambda i: (0, i, 0, 0))]
        ),
        out_shape=(
            [jax.ShapeDtypeStruct((2, N_PAD, FQ), jnp.bfloat16)] * nq
            + [jax.ShapeDtypeStruct((2, NB, 1, BLK), jnp.float32)]
        ),
    )


def _recombine(acc_refs, den_ref, q_ref, b_ref):
    q = q_ref[0, 0, 0, :]
    q2 = q_ref[1, 0, 0, :]
    den = (q * jnp.sum(den_ref[:, 0, :], axis=0)
           + q2 * jnp.sum(den_ref[:, 1, :], axis=0))
    inv = 1.0 / (den + 1e-16)
    parts = []
    for a in acc_refs:
        af = a[...].astype(jnp.float32)
        pos = af[0, 0] + af[1, 0]
        neg = af[0, 1] + af[1, 1]
        parts.append(q[:, None] * pos + q2[:, None] * neg)
    h = jnp.concatenate(parts, axis=1) * inv[:, None] + b_ref[...][None, :]
    return jnp.where(h >= 0.0, h, 0.01 * h)


def _mid_body(a0, a1, den_ref, q_ref, b_ref, w_ref, al_ref, ar_ref,
              feat_ref, el_ref, er_ref, m_ref):
    i = pl.program_id(0)
    h = _recombine((a0, a1), den_ref, q_ref, b_ref)
    feat = jnp.dot(h, w_ref[...], preferred_element_type=jnp.float32)
    feat_ref[...] = feat
    _attn_tail(i, feat, al_ref, ar_ref, el_ref, er_ref, m_ref)


def _make_tc_mid(F_in, F_out):
    nq = F_in // FQ
    return pl.pallas_call(
        _mid_body,
        grid=(NB,),
        in_specs=(
            [pl.BlockSpec((NC, 2, BLK, FQ), lambda i: (0, 0, i, 0))] * nq
            + [
                pl.BlockSpec((NW, 2, BLK), lambda i: (0, 0, i)),
                pl.BlockSpec((2, 1, 1, BLK), lambda i: (0, i, 0, 0)),
                pl.BlockSpec((F_in,), lambda i: (0,)),
                pl.BlockSpec((F_in, F_out), lambda i: (0, 0)),
                pl.BlockSpec((F_out,), lambda i: (0,)),
                pl.BlockSpec((F_out,), lambda i: (0,)),
            ]
        ),
        out_specs=[
            pl.BlockSpec((BLK, F_out), lambda i: (i, 0)),
            pl.BlockSpec((1, 1, BLK), lambda i: (i, 0, 0)),
            pl.BlockSpec((1, 1, BLK), lambda i: (i, 0, 0)),
            pl.BlockSpec((1, 2), lambda i: (0, 0), memory_space=pltpu.SMEM),
        ],
        out_shape=[
            jax.ShapeDtypeStruct((N_PAD, F_out), jnp.float32),
            jax.ShapeDtypeStruct((NB, 1, BLK), jnp.float32),
            jax.ShapeDtypeStruct((NB, 1, BLK), jnp.float32),
            jax.ShapeDtypeStruct((1, 2), jnp.float32),
        ],
    )


def _final_body(a0, den_ref, q_ref, b_ref, out_ref):
    i = pl.program_id(0)
    h = _recombine((a0,), den_ref, q_ref, b_ref)
    row = i * BLK + lax.broadcasted_iota(jnp.int32, (BLK, 1), 0)
    h = jnp.where(row < N, h, 0.0)

    @pl.when(i == 0)
    def _():
        out_ref[...] = jnp.zeros_like(out_ref)

    out_ref[...] += jnp.sum(h, axis=0, keepdims=True)

    @pl.when(i == NB - 1)
    def _():
        out_ref[...] *= jnp.float32(1.0 / N)


def _make_tc_final(F):
    nq = F // FQ
    return pl.pallas_call(
        _final_body,
        grid=(NB,),
        in_specs=(
            [pl.BlockSpec((NC, 2, BLK, FQ), lambda i: (0, 0, i, 0))] * nq
            + [
                pl.BlockSpec((NW, 2, BLK), lambda i: (0, 0, i)),
                pl.BlockSpec((2, 1, 1, BLK), lambda i: (0, i, 0, 0)),
                pl.BlockSpec((F,), lambda i: (0,)),
            ]
        ),
        out_specs=pl.BlockSpec((1, F), lambda i: (0, 0)),
        out_shape=jax.ShapeDtypeStruct((1, F), jnp.float32),
    )


def _make_sc_edge(nparts):
    mesh = plsc.VectorSubcoreMesh(core_axis_name="c", subcore_axis_name="s")

    @functools.partial(
        pl.kernel,
        out_type=(
            [jax.ShapeDtypeStruct((NC, N2, FQ), jnp.bfloat16)] * nparts
            + [jax.ShapeDtypeStruct((NW, DROW2, L), jnp.float32)]
        ),
        mesh=mesh,
        compiler_params=pltpu.CompilerParams(
            needs_layout_passes=False, use_tc_tiling_on_sc=False),
        scratch_types=[
            pltpu.VMEM((EPW,), jnp.int32),
            pltpu.VMEM((EPW,), jnp.int32),
            pltpu.VMEM((N_PAD,), jnp.float32),
            pltpu.VMEM((N_PAD,), jnp.float32),
            pltpu.VMEM((L,), jnp.float32),
            pltpu.VMEM((DROW2, L), jnp.float32),
            pltpu.VMEM((4, C, FQ), jnp.bfloat16),
            pltpu.SemaphoreType.DMA,
            pltpu.SemaphoreType.DMA,
            pltpu.SemaphoreType.DMA,
            pltpu.SemaphoreType.DMA,
            pltpu.SemaphoreType.DMA,
            pltpu.SemaphoreType.DMA,
            pltpu.SemaphoreType.DMA,
            pltpu.SemaphoreType.DMA,
            pltpu.VMEM_SHARED((N2, FQ), jnp.bfloat16),
        ],
    )
    def sc_edge(*args):
        (src1_h, dst1_h, el_h, er_h, m_h) = args[:5]
        g_hs = args[5:5 + nparts]
        z_h = args[5 + nparts]
        agg_outs = args[6 + nparts:6 + 2 * nparts]
        den_out = args[6 + 2 * nparts]
        (src1, dst1, el_v, er_v, m_v, den_v, rows_v,
         g0, g1, g2, g3, s0, s1, s2, s3, agg_sh) = args[7 + 2 * nparts:]
        gsems = (g0, g1, g2, g3)
        ssems = (s0, s1, s2, s3)

        cid = lax.axis_index("c")
        sid = lax.axis_index("s")
        wid = cid * NS + sid
        base = sid * RPT2

        pltpu.sync_copy(src1_h.at[pl.ds(wid * EPW, EPW)], src1)
        pltpu.sync_copy(dst1_h.at[pl.ds(wid * EPW, EPW)], dst1)
        pltpu.sync_copy(el_h, el_v)
        pltpu.sync_copy(er_h, er_v)
        pltpu.sync_copy(m_h, m_v)

        pltpu.sync_copy(z_h, agg_sh.at[pl.ds(base, RPT2)])

        zvec = jnp.zeros((L,), jnp.float32)

        def zden(r, carry):
            den_v[r, pl.ds(0, L)] = zvec
            return carry

        lax.fori_loop(0, DROW2, zden, 0)

        m_vec = m_v[...]

        def passa(t, carry):
            s_idx = src1[pl.ds(t * L, L)]
            d_idx = dst1[pl.ds(t * L, L)]
            els = plsc.load_gather(el_v, [s_idx])
            e = els + plsc.load_gather(er_v, [d_idx])
            neg = e < 0.0
            adj = jnp.where(neg, N_PAD, 0).astype(jnp.int32)
            si = s_idx + adj
            di = d_idx + adj
            src1[pl.ds(t * L, L)] = si
            dst1[pl.ds(t * L, L)] = di
            dl = els - m_vec
            v = jnp.exp(jnp.where(neg, 0.2 * dl, dl))
            plsc.addupdate_scatter(
                den_v, [lax.shift_right_logical(di, 4),
                        lax.bitwise_and(di, 15)], v)
            return carry

        lax.fori_loop(0, EPW // L, passa, 0)
        pltpu.sync_copy(den_v, den_out.at[wid])

        for p in range(nparts):
            plsc.subcore_barrier()

            g_h = g_hs[p]
            for b in range(4):
                pltpu.async_copy(g_h.at[src1.at[pl.ds(b * C, C)]],
                                 rows_v.at[b], gsems[b])

            def passb(i, carry):
                j4 = 4 * i
                for b in range(4):
                    j = j4 + b
                    pltpu.make_async_copy(
                        g_h.at[src1.at[pl.ds(j * C, C)]],
                        rows_v.at[b], gsems[b]).wait()
                    pltpu.async_copy(
                        rows_v.at[b], agg_sh.at[dst1.at[pl.ds(j * C, C)]],
                        ssems[b], add=True)
                for b in range(4):
                    j = j4 + b
                    pltpu.make_async_copy(
                        rows_v.at[b], agg_sh.at[dst1.at[pl.ds(j * C, C)]],
                        ssems[b]).wait()

                    @pl.when(j + 4 < NCHUNK)
                    def _():
                        pltpu.async_copy(
                            g_h.at[src1.at[pl.ds((j + 4) * C, C)]],
                            rows_v.at[b], gsems[b])
                return carry

            lax.fori_loop(0, NCHUNK // 4, passb, 0)
            jt = NCHUNK - 1
            pltpu.make_async_copy(
                g_h.at[src1.at[pl.ds(jt * C, C)]], rows_v.at[0], gsems[0]).wait()
            pltpu.sync_copy(rows_v.at[0],
                            agg_sh.at[dst1.at[pl.ds(jt * C, C)]], add=True)

            plsc.subcore_barrier()

            pltpu.sync_copy(agg_sh.at[pl.ds(base, RPT2)],
                            agg_outs[p].at[cid, pl.ds(base, RPT2)])
            if p + 1 < nparts:
                pltpu.sync_copy(z_h, agg_sh.at[pl.ds(base, RPT2)])

    return sc_edge


_tc_feat1 = _make_tc_feat(IN_F, H1_F)
_tc_prep1 = _make_tc_prep(H1_F)
_tc_mid = _make_tc_mid(H1_F, OUT_F)
_tc_prep2 = _make_tc_prep(OUT_F)
_tc_final = _make_tc_final(OUT_F)
_sc_edge1 = _make_sc_edge(H1_F // FQ)
_sc_edge2 = _make_sc_edge(OUT_F // FQ)


def kernel(x, edge_index, W1, attn_l1, attn_r1, b1, W2, attn_l2, attn_r2, b2):
    src = edge_index[0]
    dst = edge_index[1]
    x_pad = jnp.pad(x, ((0, N_PAD - N), (0, 0)))
    z = jnp.zeros((RPT2, FQ), jnp.bfloat16)

    feat1, el3, er3, m1 = _tc_feat1(x_pad, W1, attn_l1, attn_r1)
    g10, g11, q1 = _tc_prep1(feat1, el3, er3, m1)
    a0, a1, den1 = _sc_edge1(
        src, dst, el3.reshape(N_PAD), er3.reshape(N_PAD),
        jnp.full((L,), m1[0, 0], jnp.float32),
        g10.reshape(N2, FQ), g11.reshape(N2, FQ), z)

    feat2, el3b, er3b, m2 = _tc_mid(
        a0.reshape(NC, 2, N_PAD, FQ), a1.reshape(NC, 2, N_PAD, FQ),
        den1.reshape(NW, 2, N_PAD), q1, b1, W2, attn_l2, attn_r2)
    g20, q2 = _tc_prep2(feat2, el3b, er3b, m2)
    c0, den2 = _sc_edge2(
        src, dst, el3b.reshape(N_PAD), er3b.reshape(N_PAD),
        jnp.full((L,), m2[0, 0], jnp.float32),
        g20.reshape(N2, FQ), z)

    return _tc_final(c0.reshape(NC, 2, N_PAD, FQ),
                     den2.reshape(NW, 2, N_PAD), q2, b2)


# speedup vs baseline: 5.6752x; 1.0335x over previous
import functools

import jax
import jax.numpy as jnp
from jax import lax
from jax.experimental import pallas as pl
from jax.experimental.pallas import tpu as pltpu
from jax.experimental.pallas import tpu_sc as plsc

N = 10000
E = 320000
IN_F = 128
H1_F = 128
OUT_F = 64
FQ = 64

NC = 2
NS = 16
L = 16
NW = NC * NS
EPW = E // NW
C = 80
NCHUNK = EPW // C

BLK = 1024
NB = 10
N_PAD = NB * BLK
N2 = 2 * N_PAD
RPT2 = N2 // NS
DROW2 = N2 // L

_NEG_INF = -3.0e38


def _attn_tail(i, feat, al_ref, ar_ref, el_ref, er_ref, m_ref):
    el = jnp.sum(feat * al_ref[...][None, :], axis=1)
    er = jnp.sum(feat * ar_ref[...][None, :], axis=1)
    el_ref[0, 0, :] = el
    er_ref[0, 0, :] = er

    @pl.when(i == 0)
    def _():
        m_ref[0, 0] = _NEG_INF
        m_ref[0, 1] = _NEG_INF

    m_ref[0, 0] = jnp.maximum(m_ref[0, 0], jnp.max(el))
    m_ref[0, 1] = jnp.maximum(m_ref[0, 1], jnp.max(er))


def _feat_body(h_ref, w_ref, al_ref, ar_ref, feat_ref, el_ref, er_ref, m_ref):
    i = pl.program_id(0)
    feat = jnp.dot(h_ref[...], w_ref[...], preferred_element_type=jnp.float32)
    feat_ref[...] = feat
    _attn_tail(i, feat, al_ref, ar_ref, el_ref, er_ref, m_ref)


def _make_tc_feat(F_in, F_out):
    return pl.pallas_call(
        _feat_body,
        grid=(NB,),
        in_specs=[
            pl.BlockSpec((BLK, F_in), lambda i: (i, 0)),
            pl.BlockSpec((F_in, F_out), lambda i: (0, 0)),
            pl.BlockSpec((F_out,), lambda i: (0,)),
            pl.BlockSpec((F_out,), lambda i: (0,)),
        ],
        out_specs=[
            pl.BlockSpec((BLK, F_out), lambda i: (i, 0)),
            pl.BlockSpec((1, 1, BLK), lambda i: (i, 0, 0)),
            pl.BlockSpec((1, 1, BLK), lambda i: (i, 0, 0)),
            pl.BlockSpec((1, 2), lambda i: (0, 0), memory_space=pltpu.SMEM),
        ],
        out_shape=[
            jax.ShapeDtypeStruct((N_PAD, F_out), jnp.float32),
            jax.ShapeDtypeStruct((NB, 1, BLK), jnp.float32),
            jax.ShapeDtypeStruct((NB, 1, BLK), jnp.float32),
            jax.ShapeDtypeStruct((1, 2), jnp.float32),
        ],
    )


def _prep_body(feat_ref, el_ref, er_ref, m_ref, *out_refs):
    nq = len(out_refs) - 1
    g_refs = out_refs[:nq]
    q_ref = out_refs[nq]
    max_el = m_ref[0, 0]
    max_er = m_ref[0, 1]
    mt = max_el + max_er
    el = el_ref[0, 0, :]
    er = er_ref[0, 0, :]
    P = jnp.exp(el - max_el)
    P2 = jnp.exp(0.2 * (el - max_el))
    Q = jnp.exp(er - max_er)
    Q2 = jnp.exp(0.2 * (er - max_er) - 0.8 * mt)
    q_ref[0, 0, 0, :] = Q
    q_ref[1, 0, 0, :] = Q2
    feat = feat_ref[...]
    for q in range(nq):
        cols = feat[:, q * FQ:(q + 1) * FQ]
        g_refs[q][0] = (cols * P[:, None]).astype(jnp.bfloat16)
        g_refs[q][1] = (cols * P2[:, None]).astype(jnp.bfloat16)


def _make_tc_prep(F):
    nq = F // FQ
    return pl.pallas_call(
        _prep_body,
        grid=(NB,),
        in_specs=[
            pl.BlockSpec((BLK, F), lambda i: (i, 0)),
            pl.BlockSpec((1, 1, BLK), lambda i: (i, 0, 0)),
            pl.BlockSpec((1, 1, BLK), lambda i: (i, 0, 0)),
            pl.BlockSpec((1, 2), lambda i: (0, 0), memory_space=pltpu.SMEM),
        ],
        out_specs=(
            [pl.BlockSpec((2, BLK, FQ), lambda i: (0, i, 0))] * nq
            + [pl.BlockSpec((2, 1, 1, BLK), lambda i: (0, i, 0, 0))]
        ),
        out_shape=(
            [jax.ShapeDtypeStruct((2, N_PAD, FQ), jnp.bfloat16)] * nq
            + [jax.ShapeDtypeStruct((2, NB, 1, BLK), jnp.float32)]
        ),
    )


def _recombine(acc_refs, den_ref, q_ref, b_ref):
    q = q_ref[0, 0, 0, :]
    q2 = q_ref[1, 0, 0, :]
    den = (q * jnp.sum(den_ref[:, 0, :], axis=0)
           + q2 * jnp.sum(den_ref[:, 1, :], axis=0))
    inv = 1.0 / (den + 1e-16)
    parts = []
    for a in acc_refs:
        af = a[...].astype(jnp.float32)
        pos = af[0, 0] + af[1, 0]
        neg = af[0, 1] + af[1, 1]
        parts.append(q[:, None] * pos + q2[:, None] * neg)
    h = jnp.concatenate(parts, axis=1) * inv[:, None] + b_ref[...][None, :]
    return jnp.where(h >= 0.0, h, 0.01 * h)


def _mid_body(a0, a1, den_ref, q_ref, b_ref, w_ref, al_ref, ar_ref,
              feat_ref, el_ref, er_ref, m_ref):
    i = pl.program_id(0)
    h = _recombine((a0, a1), den_ref, q_ref, b_ref)
    feat = jnp.dot(h, w_ref[...], preferred_element_type=jnp.float32)
    feat_ref[...] = feat
    _attn_tail(i, feat, al_ref, ar_ref, el_ref, er_ref, m_ref)


def _make_tc_mid(F_in, F_out):
    nq = F_in // FQ
    return pl.pallas_call(
        _mid_body,
        grid=(NB,),
        in_specs=(
            [pl.BlockSpec((NC, 2, BLK, FQ), lambda i: (0, 0, i, 0))] * nq
            + [
                pl.BlockSpec((NW, 2, BLK), lambda i: (0, 0, i)),
                pl.BlockSpec((2, 1, 1, BLK), lambda i: (0, i, 0, 0)),
                pl.BlockSpec((F_in,), lambda i: (0,)),
                pl.BlockSpec((F_in, F_out), lambda i: (0, 0)),
                pl.BlockSpec((F_out,), lambda i: (0,)),
                pl.BlockSpec((F_out,), lambda i: (0,)),
            ]
        ),
        out_specs=[
            pl.BlockSpec((BLK, F_out), lambda i: (i, 0)),
            pl.BlockSpec((1, 1, BLK), lambda i: (i, 0, 0)),
            pl.BlockSpec((1, 1, BLK), lambda i: (i, 0, 0)),
            pl.BlockSpec((1, 2), lambda i: (0, 0), memory_space=pltpu.SMEM),
        ],
        out_shape=[
            jax.ShapeDtypeStruct((N_PAD, F_out), jnp.float32),
            jax.ShapeDtypeStruct((NB, 1, BLK), jnp.float32),
            jax.ShapeDtypeStruct((NB, 1, BLK), jnp.float32),
            jax.ShapeDtypeStruct((1, 2), jnp.float32),
        ],
    )


def _final_body(a0, den_ref, q_ref, b_ref, out_ref):
    i = pl.program_id(0)
    h = _recombine((a0,), den_ref, q_ref, b_ref)
    row = i * BLK + lax.broadcasted_iota(jnp.int32, (BLK, 1), 0)
    h = jnp.where(row < N, h, 0.0)

    @pl.when(i == 0)
    def _():
        out_ref[...] = jnp.zeros_like(out_ref)

    out_ref[...] += jnp.sum(h, axis=0, keepdims=True)

    @pl.when(i == NB - 1)
    def _():
        out_ref[...] *= jnp.float32(1.0 / N)


def _make_tc_final(F):
    nq = F // FQ
    return pl.pallas_call(
        _final_body,
        grid=(NB,),
        in_specs=(
            [pl.BlockSpec((NC, 2, BLK, FQ), lambda i: (0, 0, i, 0))] * nq
            + [
                pl.BlockSpec((NW, 2, BLK), lambda i: (0, 0, i)),
                pl.BlockSpec((2, 1, 1, BLK), lambda i: (0, i, 0, 0)),
                pl.BlockSpec((F,), lambda i: (0,)),
            ]
        ),
        out_specs=pl.BlockSpec((1, F), lambda i: (0, 0)),
        out_shape=jax.ShapeDtypeStruct((1, F), jnp.float32),
    )


def _make_sc_edge(nparts):
    mesh = plsc.VectorSubcoreMesh(core_axis_name="c", subcore_axis_name="s")

    @functools.partial(
        pl.kernel,
        out_type=(
            [jax.ShapeDtypeStruct((NC, N2, FQ), jnp.bfloat16)] * nparts
            + [jax.ShapeDtypeStruct((NW, DROW2, L), jnp.float32)]
        ),
        mesh=mesh,
        compiler_params=pltpu.CompilerParams(
            needs_layout_passes=False, use_tc_tiling_on_sc=False),
        scratch_types=[
            pltpu.VMEM((EPW,), jnp.int32),
            pltpu.VMEM((EPW,), jnp.int32),
            pltpu.VMEM((N_PAD,), jnp.float32),
            pltpu.VMEM((N_PAD,), jnp.float32),
            pltpu.VMEM((L,), jnp.float32),
            pltpu.VMEM((DROW2, L), jnp.float32),
            pltpu.VMEM((4, C, FQ), jnp.bfloat16),
            pltpu.SemaphoreType.DMA,
            pltpu.SemaphoreType.DMA,
            pltpu.SemaphoreType.DMA,
            pltpu.SemaphoreType.DMA,
            pltpu.SemaphoreType.DMA,
            pltpu.SemaphoreType.DMA,
            pltpu.SemaphoreType.DMA,
            pltpu.SemaphoreType.DMA,
            pltpu.VMEM_SHARED((N2, FQ), jnp.bfloat16),
        ],
    )
    def sc_edge(*args):
        (ei_h, el_h, er_h, m_h) = args[:4]
        g_hs = args[4:4 + nparts]
        agg_outs = args[4 + nparts:4 + 2 * nparts]
        den_out = args[4 + 2 * nparts]
        (src1, dst1, el_v, er_v, m_v, den_v, rows_v,
         g0, g1, g2, g3, s0, s1, s2, s3, agg_sh) = args[5 + 2 * nparts:]
        gsems = (g0, g1, g2, g3)
        ssems = (s0, s1, s2, s3)

        cid = lax.axis_index("c")
        sid = lax.axis_index("s")
        wid = cid * NS + sid
        base = sid * RPT2

        pltpu.sync_copy(ei_h.at[0, pl.ds(wid * EPW, EPW)], src1)
        pltpu.sync_copy(ei_h.at[1, pl.ds(wid * EPW, EPW)], dst1)
        pltpu.sync_copy(el_h, el_v)
        pltpu.sync_copy(er_h, er_v)
        pltpu.sync_copy(m_h, m_v)

        zvec = jnp.zeros((L,), jnp.float32)
        zvec_b = jnp.zeros((2 * L,), jnp.bfloat16)

        def _zero_acc_slice():
            def zrow(r, carry):
                rows_v[0, r, pl.ds(0, 2 * L)] = zvec_b
                rows_v[0, r, pl.ds(2 * L, 2 * L)] = zvec_b
                return carry

            lax.fori_loop(0, C, zrow, 0)
            for t in range(RPT2 // C):
                pltpu.sync_copy(rows_v.at[0],
                                agg_sh.at[pl.ds(base + t * C, C)])

        _zero_acc_slice()

        def zden(r, carry):
            den_v[r, pl.ds(0, L)] = zvec
            return carry

        lax.fori_loop(0, DROW2, zden, 0)

        m_vec = m_v[...]

        def passa(t, carry):
            s_idx = src1[pl.ds(t * L, L)]
            d_idx = dst1[pl.ds(t * L, L)]
            els = plsc.load_gather(el_v, [s_idx])
            e = els + plsc.load_gather(er_v, [d_idx])
            neg = e < 0.0
            adj = jnp.where(neg, N_PAD, 0).astype(jnp.int32)
            si = s_idx + adj
            di = d_idx + adj
            src1[pl.ds(t * L, L)] = si
            dst1[pl.ds(t * L, L)] = di
            dl = els - m_vec
            v = jnp.exp(jnp.where(neg, 0.2 * dl, dl))
            plsc.addupdate_scatter(
                den_v, [lax.shift_right_logical(di, 4),
                        lax.bitwise_and(di, 15)], v)
            return carry

        lax.fori_loop(0, EPW // L, passa, 0)
        pltpu.sync_copy(den_v, den_out.at[wid])

        for p in range(nparts):
            plsc.subcore_barrier()

            g_h = g_hs[p]
            for b in range(4):
                pltpu.async_copy(g_h.at[src1.at[pl.ds(b * C, C)]],
                                 rows_v.at[b], gsems[b])

            def passb(i, carry):
                j4 = 4 * i
                for b in range(4):
                    j = j4 + b
                    pltpu.make_async_copy(
                        g_h.at[src1.at[pl.ds(j * C, C)]],
                        rows_v.at[b], gsems[b]).wait()
                    pltpu.async_copy(
                        rows_v.at[b], agg_sh.at[dst1.at[pl.ds(j * C, C)]],
                        ssems[b], add=True)
                for b in range(4):
                    j = j4 + b
                    pltpu.make_async_copy(
                        rows_v.at[b], agg_sh.at[dst1.at[pl.ds(j * C, C)]],
                        ssems[b]).wait()

                    @pl.when(j + 4 < NCHUNK)
                    def _():
                        pltpu.async_copy(
                            g_h.at[src1.at[pl.ds((j + 4) * C, C)]],
                            rows_v.at[b], gsems[b])
                return carry

            lax.fori_loop(0, NCHUNK // 4, passb, 0)
            jt = NCHUNK - 1
            pltpu.make_async_copy(
                g_h.at[src1.at[pl.ds(jt * C, C)]], rows_v.at[0], gsems[0]).wait()
            pltpu.sync_copy(rows_v.at[0],
                            agg_sh.at[dst1.at[pl.ds(jt * C, C)]], add=True)

            plsc.subcore_barrier()

            pltpu.sync_copy(agg_sh.at[pl.ds(base, RPT2)],
                            agg_outs[p].at[cid, pl.ds(base, RPT2)])
            if p + 1 < nparts:
                _zero_acc_slice()

    return sc_edge


_tc_feat1 = _make_tc_feat(IN_F, H1_F)
_tc_prep1 = _make_tc_prep(H1_F)
_tc_mid = _make_tc_mid(H1_F, OUT_F)
_tc_prep2 = _make_tc_prep(OUT_F)
_tc_final = _make_tc_final(OUT_F)
_sc_edge1 = _make_sc_edge(H1_F // FQ)
_sc_edge2 = _make_sc_edge(OUT_F // FQ)


def kernel(x, edge_index, W1, attn_l1, attn_r1, b1, W2, attn_l2, attn_r2, b2):
    x_pad = jnp.pad(x, ((0, N_PAD - N), (0, 0)))

    feat1, el3, er3, m1 = _tc_feat1(x_pad, W1, attn_l1, attn_r1)
    g10, g11, q1 = _tc_prep1(feat1, el3, er3, m1)
    a0, a1, den1 = _sc_edge1(
        edge_index, el3.reshape(N_PAD), er3.reshape(N_PAD),
        jnp.full((L,), m1[0, 0], jnp.float32),
        g10.reshape(N2, FQ), g11.reshape(N2, FQ))

    feat2, el3b, er3b, m2 = _tc_mid(
        a0.reshape(NC, 2, N_PAD, FQ), a1.reshape(NC, 2, N_PAD, FQ),
        den1.reshape(NW, 2, N_PAD), q1, b1, W2, attn_l2, attn_r2)
    g20, q2 = _tc_prep2(feat2, el3b, er3b, m2)
    c0, den2 = _sc_edge2(
        edge_index, el3b.reshape(N_PAD), er3b.reshape(N_PAD),
        jnp.full((L,), m2[0, 0], jnp.float32),
        g20.reshape(N2, FQ))

    return _tc_final(c0.reshape(NC, 2, N_PAD, FQ),
                     den2.reshape(NW, 2, N_PAD), q2, b2)


# speedup vs baseline: 5.6945x; 1.0034x over previous
import functools

import jax
import jax.numpy as jnp
from jax import lax
from jax.experimental import pallas as pl
from jax.experimental.pallas import tpu as pltpu
from jax.experimental.pallas import tpu_sc as plsc

N = 10000
E = 320000
IN_F = 128
H1_F = 128
OUT_F = 64
FQ = 64

NC = 2
NS = 16
L = 16
NW = NC * NS
EPW = E // NW
C = 80
NCHUNK = EPW // C

BLK = 1024
NB = 10
N_PAD = NB * BLK
N2 = 2 * N_PAD
RPT2 = N2 // NS
DROW2 = N2 // L

_NEG_INF = -3.0e38


def _phase2_tables(el, er, m_ref, feat, g_refs, q_ref, el_ref, er_ref):
    max_el = m_ref[0, 0]
    max_er = m_ref[0, 1]
    mt = max_el + max_er
    P = jnp.exp(el - max_el)
    P2 = jnp.exp(0.2 * (el - max_el))
    q_ref[0, 0, 0, :] = jnp.exp(er - max_er)
    q_ref[1, 0, 0, :] = jnp.exp(0.2 * (er - max_er) - 0.8 * mt)
    el_ref[0, 0, :] = el - max_el
    er_ref[0, 0, :] = er + max_el
    for q in range(len(g_refs)):
        cols = feat[:, q * FQ:(q + 1) * FQ]
        g_refs[q][0] = (cols * P[:, None]).astype(jnp.bfloat16)
        g_refs[q][1] = (cols * P2[:, None]).astype(jnp.bfloat16)


def _featprep_tail(i, i0, feat_new, al_ref, ar_ref, g_refs, q_ref,
                   el_ref, er_ref, feat_s, el_s, er_s, m_s):
    @pl.when(i == 0)
    def _():
        m_s[0, 0] = _NEG_INF
        m_s[0, 1] = _NEG_INF

    @pl.when(i < NB)
    def _():
        feat = feat_new()
        feat_s[pl.ds(i0 * BLK, BLK), :] = feat
        el = jnp.sum(feat * al_ref[...][None, :], axis=1)
        er = jnp.sum(feat * ar_ref[...][None, :], axis=1)
        el_s[pl.ds(i0, 1), :] = el[None, :]
        er_s[pl.ds(i0, 1), :] = er[None, :]
        m_s[0, 0] = jnp.maximum(m_s[0, 0], jnp.max(el))
        m_s[0, 1] = jnp.maximum(m_s[0, 1], jnp.max(er))

    @pl.when(i >= NB)
    def _():
        feat = feat_s[pl.ds(i0 * BLK, BLK), :]
        el = el_s[pl.ds(i0, 1), :][0]
        er = er_s[pl.ds(i0, 1), :][0]
        _phase2_tables(el, er, m_s, feat, g_refs, q_ref, el_ref, er_ref)


def _make_tc_featprep(F_in, F_out):
    nq = F_out // FQ

    def body(x_ref, w_ref, al_ref, ar_ref, *rest):
        g_refs = rest[:nq]
        q_ref, el_ref, er_ref, feat_s, el_s, er_s, m_s = rest[nq:]
        i = pl.program_id(0)
        i0 = i % NB
        _featprep_tail(
            i, i0,
            lambda: jnp.dot(x_ref[...], w_ref[...],
                            preferred_element_type=jnp.float32),
            al_ref, ar_ref, g_refs, q_ref, el_ref, er_ref,
            feat_s, el_s, er_s, m_s)

    return pl.pallas_call(
        body,
        grid=(2 * NB,),
        in_specs=[
            pl.BlockSpec((BLK, F_in), lambda i: (i % NB, 0)),
            pl.BlockSpec((F_in, F_out), lambda i: (0, 0)),
            pl.BlockSpec((F_out,), lambda i: (0,)),
            pl.BlockSpec((F_out,), lambda i: (0,)),
        ],
        out_specs=(
            [pl.BlockSpec((2, BLK, FQ), lambda i: (0, i % NB, 0))] * nq
            + [
                pl.BlockSpec((2, 1, 1, BLK), lambda i: (0, i % NB, 0, 0)),
                pl.BlockSpec((1, 1, BLK), lambda i: (i % NB, 0, 0)),
                pl.BlockSpec((1, 1, BLK), lambda i: (i % NB, 0, 0)),
            ]
        ),
        out_shape=(
            [jax.ShapeDtypeStruct((2, N_PAD, FQ), jnp.bfloat16)] * nq
            + [
                jax.ShapeDtypeStruct((2, NB, 1, BLK), jnp.float32),
                jax.ShapeDtypeStruct((NB, 1, BLK), jnp.float32),
                jax.ShapeDtypeStruct((NB, 1, BLK), jnp.float32),
            ]
        ),
        scratch_shapes=[
            pltpu.VMEM((N_PAD, F_out), jnp.float32),
            pltpu.VMEM((NB, BLK), jnp.float32),
            pltpu.VMEM((NB, BLK), jnp.float32),
            pltpu.SMEM((1, 2), jnp.float32),
        ],
    )


def _recombine(acc_refs, den_ref, q_ref, b_ref):
    q = q_ref[0, 0, 0, :]
    q2 = q_ref[1, 0, 0, :]
    den = (q * jnp.sum(den_ref[:, 0, :], axis=0)
           + q2 * jnp.sum(den_ref[:, 1, :], axis=0))
    inv = 1.0 / (den + 1e-16)
    parts = []
    for a in acc_refs:
        af = a[...].astype(jnp.float32)
        pos = af[0, 0] + af[1, 0]
        neg = af[0, 1] + af[1, 1]
        parts.append(q[:, None] * pos + q2[:, None] * neg)
    h = jnp.concatenate(parts, axis=1) * inv[:, None] + b_ref[...][None, :]
    return jnp.where(h >= 0.0, h, 0.01 * h)


def _make_tc_midprep(F_in, F_out):
    nq_in = F_in // FQ
    nq = F_out // FQ

    def body(*args):
        a_refs = args[:nq_in]
        den_ref, q1_ref, b_ref, w_ref, al_ref, ar_ref = args[nq_in:nq_in + 6]
        rest = args[nq_in + 6:]
        g_refs = rest[:nq]
        q_ref, el_ref, er_ref, feat_s, el_s, er_s, m_s = rest[nq:]
        i = pl.program_id(0)
        i0 = i % NB

        def feat_new():
            h = _recombine(a_refs, den_ref, q1_ref, b_ref)
            return jnp.dot(h, w_ref[...], preferred_element_type=jnp.float32)

        _featprep_tail(i, i0, feat_new, al_ref, ar_ref, g_refs, q_ref,
                       el_ref, er_ref, feat_s, el_s, er_s, m_s)

    return pl.pallas_call(
        body,
        grid=(2 * NB,),
        in_specs=(
            [pl.BlockSpec((NC, 2, BLK, FQ), lambda i: (0, 0, i % NB, 0))]
            * nq_in
            + [
                pl.BlockSpec((NW, 2, BLK), lambda i: (0, 0, i % NB)),
                pl.BlockSpec((2, 1, 1, BLK), lambda i: (0, i % NB, 0, 0)),
                pl.BlockSpec((F_in,), lambda i: (0,)),
                pl.BlockSpec((F_in, F_out), lambda i: (0, 0)),
                pl.BlockSpec((F_out,), lambda i: (0,)),
                pl.BlockSpec((F_out,), lambda i: (0,)),
            ]
        ),
        out_specs=(
            [pl.BlockSpec((2, BLK, FQ), lambda i: (0, i % NB, 0))] * nq
            + [
                pl.BlockSpec((2, 1, 1, BLK), lambda i: (0, i % NB, 0, 0)),
                pl.BlockSpec((1, 1, BLK), lambda i: (i % NB, 0, 0)),
                pl.BlockSpec((1, 1, BLK), lambda i: (i % NB, 0, 0)),
            ]
        ),
        out_shape=(
            [jax.ShapeDtypeStruct((2, N_PAD, FQ), jnp.bfloat16)] * nq
            + [
                jax.ShapeDtypeStruct((2, NB, 1, BLK), jnp.float32),
                jax.ShapeDtypeStruct((NB, 1, BLK), jnp.float32),
                jax.ShapeDtypeStruct((NB, 1, BLK), jnp.float32),
            ]
        ),
        scratch_shapes=[
            pltpu.VMEM((N_PAD, F_out), jnp.float32),
            pltpu.VMEM((NB, BLK), jnp.float32),
            pltpu.VMEM((NB, BLK), jnp.float32),
            pltpu.SMEM((1, 2), jnp.float32),
        ],
    )


def _final_body(a0, den_ref, q_ref, b_ref, out_ref):
    i = pl.program_id(0)
    h = _recombine((a0,), den_ref, q_ref, b_ref)
    row = i * BLK + lax.broadcasted_iota(jnp.int32, (BLK, 1), 0)
    h = jnp.where(row < N, h, 0.0)

    @pl.when(i == 0)
    def _():
        out_ref[...] = jnp.zeros_like(out_ref)

    out_ref[...] += jnp.sum(h, axis=0, keepdims=True)

    @pl.when(i == NB - 1)
    def _():
        out_ref[...] *= jnp.float32(1.0 / N)


def _make_tc_final(F):
    nq = F // FQ
    return pl.pallas_call(
        _final_body,
        grid=(NB,),
        in_specs=(
            [pl.BlockSpec((NC, 2, BLK, FQ), lambda i: (0, 0, i, 0))] * nq
            + [
                pl.BlockSpec((NW, 2, BLK), lambda i: (0, 0, i)),
                pl.BlockSpec((2, 1, 1, BLK), lambda i: (0, i, 0, 0)),
                pl.BlockSpec((F,), lambda i: (0,)),
            ]
        ),
        out_specs=pl.BlockSpec((1, F), lambda i: (0, 0)),
        out_shape=jax.ShapeDtypeStruct((1, F), jnp.float32),
    )


def _make_sc_edge(nparts):
    mesh = plsc.VectorSubcoreMesh(core_axis_name="c", subcore_axis_name="s")

    @functools.partial(
        pl.kernel,
        out_type=(
            [jax.ShapeDtypeStruct((NC, N2, FQ), jnp.bfloat16)] * nparts
            + [jax.ShapeDtypeStruct((NW, DROW2, L), jnp.float32)]
        ),
        mesh=mesh,
        compiler_params=pltpu.CompilerParams(
            needs_layout_passes=False, use_tc_tiling_on_sc=False),
        scratch_types=[
            pltpu.VMEM((EPW,), jnp.int32),
            pltpu.VMEM((EPW,), jnp.int32),
            pltpu.VMEM((N_PAD,), jnp.float32),
            pltpu.VMEM((N_PAD,), jnp.float32),
            pltpu.VMEM((DROW2, L), jnp.float32),
            pltpu.VMEM((4, C, FQ), jnp.bfloat16),
            pltpu.SemaphoreType.DMA,
            pltpu.SemaphoreType.DMA,
            pltpu.SemaphoreType.DMA,
            pltpu.SemaphoreType.DMA,
            pltpu.SemaphoreType.DMA,
            pltpu.SemaphoreType.DMA,
            pltpu.SemaphoreType.DMA,
            pltpu.SemaphoreType.DMA,
            pltpu.VMEM_SHARED((N2, FQ), jnp.bfloat16),
        ],
    )
    def sc_edge(*args):
        (ei_h, el_h, er_h) = args[:3]
        g_hs = args[3:3 + nparts]
        agg_outs = args[3 + nparts:3 + 2 * nparts]
        den_out = args[3 + 2 * nparts]
        (src1, dst1, el_v, er_v, den_v, rows_v,
         g0, g1, g2, g3, s0, s1, s2, s3, agg_sh) = args[4 + 2 * nparts:]
        gsems = (g0, g1, g2, g3)
        ssems = (s0, s1, s2, s3)

        cid = lax.axis_index("c")
        sid = lax.axis_index("s")
        wid = cid * NS + sid
        base = sid * RPT2

        pltpu.sync_copy(ei_h.at[0, pl.ds(wid * EPW, EPW)], src1)
        pltpu.sync_copy(ei_h.at[1, pl.ds(wid * EPW, EPW)], dst1)
        pltpu.sync_copy(el_h, el_v)
        pltpu.sync_copy(er_h, er_v)

        zvec = jnp.zeros((L,), jnp.float32)
        zvec_b = jnp.zeros((2 * L,), jnp.bfloat16)

        def _zero_acc_slice():
            def zrow(r, carry):
                rows_v[0, r, pl.ds(0, 2 * L)] = zvec_b
                rows_v[0, r, pl.ds(2 * L, 2 * L)] = zvec_b
                return carry

            lax.fori_loop(0, C, zrow, 0)
            for t in range(RPT2 // C):
                pltpu.sync_copy(rows_v.at[0],
                                agg_sh.at[pl.ds(base + t * C, C)])

        _zero_acc_slice()

        def zden(r, carry):
            den_v[r, pl.ds(0, L)] = zvec
            return carry

        lax.fori_loop(0, DROW2, zden, 0)

        def passa(t, carry):
            s_idx = src1[pl.ds(t * L, L)]
            d_idx = dst1[pl.ds(t * L, L)]
            els = plsc.load_gather(el_v, [s_idx])
            e = els + plsc.load_gather(er_v, [d_idx])
            neg = e < 0.0
            adj = jnp.where(neg, N_PAD, 0).astype(jnp.int32)
            si = s_idx + adj
            di = d_idx + adj
            src1[pl.ds(t * L, L)] = si
            dst1[pl.ds(t * L, L)] = di
            v = jnp.exp(jnp.where(neg, 0.2 * els, els))
            plsc.addupdate_scatter(
                den_v, [lax.shift_right_logical(di, 4),
                        lax.bitwise_and(di, 15)], v)
            return carry

        lax.fori_loop(0, EPW // L, passa, 0)
        pltpu.sync_copy(den_v, den_out.at[wid])

        for p in range(nparts):
            plsc.subcore_barrier()

            g_h = g_hs[p]
            for b in range(4):
                pltpu.async_copy(g_h.at[src1.at[pl.ds(b * C, C)]],
                                 rows_v.at[b], gsems[b])

            def passb(i, carry):
                j4 = 4 * i
                for b in range(4):
                    j = j4 + b
                    pltpu.make_async_copy(
                        g_h.at[src1.at[pl.ds(j * C, C)]],
                        rows_v.at[b], gsems[b]).wait()
                    pltpu.async_copy(
                        rows_v.at[b], agg_sh.at[dst1.at[pl.ds(j * C, C)]],
                        ssems[b], add=True)
                for b in range(4):
                    j = j4 + b
                    pltpu.make_async_copy(
                        rows_v.at[b], agg_sh.at[dst1.at[pl.ds(j * C, C)]],
                        ssems[b]).wait()

                    @pl.when(j + 4 < NCHUNK)
                    def _():
                        pltpu.async_copy(
                            g_h.at[src1.at[pl.ds((j + 4) * C, C)]],
                            rows_v.at[b], gsems[b])
                return carry

            lax.fori_loop(0, NCHUNK // 4, passb, 0)
            jt = NCHUNK - 1
            pltpu.make_async_copy(
                g_h.at[src1.at[pl.ds(jt * C, C)]], rows_v.at[0], gsems[0]).wait()
            pltpu.sync_copy(rows_v.at[0],
                            agg_sh.at[dst1.at[pl.ds(jt * C, C)]], add=True)

            plsc.subcore_barrier()

            pltpu.sync_copy(agg_sh.at[pl.ds(base, RPT2)],
                            agg_outs[p].at[cid, pl.ds(base, RPT2)])
            if p + 1 < nparts:
                _zero_acc_slice()

    return sc_edge


_tc_featprep1 = _make_tc_featprep(IN_F, H1_F)
_tc_midprep = _make_tc_midprep(H1_F, OUT_F)
_tc_final = _make_tc_final(OUT_F)
_sc_edge1 = _make_sc_edge(H1_F // FQ)
_sc_edge2 = _make_sc_edge(OUT_F // FQ)


def kernel(x, edge_index, W1, attn_l1, attn_r1, b1, W2, attn_l2, attn_r2, b2):
    x_pad = jnp.pad(x, ((0, N_PAD - N), (0, 0)))

    g10, g11, q1, el3, er3 = _tc_featprep1(x_pad, W1, attn_l1, attn_r1)
    a0, a1, den1 = _sc_edge1(
        edge_index, el3.reshape(N_PAD), er3.reshape(N_PAD),
        g10.reshape(N2, FQ), g11.reshape(N2, FQ))

    g20, q2, el3b, er3b = _tc_midprep(
        a0.reshape(NC, 2, N_PAD, FQ), a1.reshape(NC, 2, N_PAD, FQ),
        den1.reshape(NW, 2, N_PAD), q1, b1, W2, attn_l2, attn_r2)
    c0, den2 = _sc_edge2(
        edge_index, el3b.reshape(N_PAD), er3b.reshape(N_PAD),
        g20.reshape(N2, FQ))

    return _tc_final(c0.reshape(NC, 2, N_PAD, FQ),
                     den2.reshape(NW, 2, N_PAD), q2, b2)


# speedup vs baseline: 6.6381x; 1.1657x over previous
import functools

import jax
import jax.numpy as jnp
from jax import lax
from jax.experimental import pallas as pl
from jax.experimental.pallas import tpu as pltpu
from jax.experimental.pallas import tpu_sc as plsc

N = 10000
E = 320000
IN_F = 128
H1_F = 128
OUT_F = 64
FQ = 64

NC = 2
NS = 16
L = 16
NW = NC * NS
EPW = E // NW
C = 80
NCHUNK = EPW // C

BLK = 1024
NB = 10
N_PAD = NB * BLK
N2 = 2 * N_PAD
RPT2 = N2 // NS
DROW2 = N2 // L

_NEG_INF = -3.0e38


def _phase2_tables(el, er, m_ref, feat, g_refs, q_ref, el_ref, er_ref):
    max_el = m_ref[0, 0]
    max_er = m_ref[0, 1]
    mt = max_el + max_er
    P = jnp.exp(el - max_el)
    P2 = jnp.exp(0.2 * (el - max_el))
    q_ref[0, 0, 0, :] = jnp.exp(er - max_er)
    q_ref[1, 0, 0, :] = jnp.exp(0.2 * (er - max_er) - 0.8 * mt)
    el_ref[...] = el - max_el
    er_ref[...] = er + max_el
    for q in range(len(g_refs)):
        cols = feat[:, q * FQ:(q + 1) * FQ]
        g_refs[q][0] = (cols * P[:, None]).astype(jnp.bfloat16)
        g_refs[q][1] = (cols * P2[:, None]).astype(jnp.bfloat16)


def _featprep_tail(i, i0, feat_new, al_ref, ar_ref, g_refs, q_ref,
                   el_ref, er_ref, feat_s, el_s, er_s, m_s):
    @pl.when(i == 0)
    def _():
        m_s[0, 0] = _NEG_INF
        m_s[0, 1] = _NEG_INF

    @pl.when(i < NB)
    def _():
        feat = feat_new()
        feat_s[pl.ds(i0 * BLK, BLK), :] = feat
        el = jnp.sum(feat * al_ref[...][None, :], axis=1)
        er = jnp.sum(feat * ar_ref[...][None, :], axis=1)
        el_s[pl.ds(i0, 1), :] = el[None, :]
        er_s[pl.ds(i0, 1), :] = er[None, :]
        m_s[0, 0] = jnp.maximum(m_s[0, 0], jnp.max(el))
        m_s[0, 1] = jnp.maximum(m_s[0, 1], jnp.max(er))

    @pl.when(i >= NB)
    def _():
        feat = feat_s[pl.ds(i0 * BLK, BLK), :]
        el = el_s[pl.ds(i0, 1), :][0]
        er = er_s[pl.ds(i0, 1), :][0]
        _phase2_tables(el, er, m_s, feat, g_refs, q_ref, el_ref, er_ref)


def _make_tc_featprep(F_in, F_out):
    nq = F_out // FQ

    def body(x_ref, w_ref, al_ref, ar_ref, *rest):
        g_refs = rest[:nq]
        q_ref, el_ref, er_ref, feat_s, el_s, er_s, m_s = rest[nq:]
        i = pl.program_id(0)
        i0 = i % NB
        _featprep_tail(
            i, i0,
            lambda: jnp.dot(x_ref[...], w_ref[...],
                            preferred_element_type=jnp.float32),
            al_ref, ar_ref, g_refs, q_ref, el_ref, er_ref,
            feat_s, el_s, er_s, m_s)

    return pl.pallas_call(
        body,
        grid=(2 * NB,),
        in_specs=[
            pl.BlockSpec((BLK, F_in), lambda i: (i % NB, 0)),
            pl.BlockSpec((F_in, F_out), lambda i: (0, 0)),
            pl.BlockSpec((F_out,), lambda i: (0,)),
            pl.BlockSpec((F_out,), lambda i: (0,)),
        ],
        out_specs=(
            [pl.BlockSpec((2, BLK, FQ), lambda i: (0, i % NB, 0))] * nq
            + [
                pl.BlockSpec((2, 1, 1, BLK), lambda i: (0, i % NB, 0, 0)),
                pl.BlockSpec((BLK,), lambda i: (i % NB,)),
                pl.BlockSpec((BLK,), lambda i: (i % NB,)),
            ]
        ),
        out_shape=(
            [jax.ShapeDtypeStruct((2, N_PAD, FQ), jnp.bfloat16)] * nq
            + [
                jax.ShapeDtypeStruct((2, NB, 1, BLK), jnp.float32),
                jax.ShapeDtypeStruct((N_PAD,), jnp.float32),
                jax.ShapeDtypeStruct((N_PAD,), jnp.float32),
            ]
        ),
        scratch_shapes=[
            pltpu.VMEM((N_PAD, F_out), jnp.float32),
            pltpu.VMEM((NB, BLK), jnp.float32),
            pltpu.VMEM((NB, BLK), jnp.float32),
            pltpu.SMEM((1, 2), jnp.float32),
        ],
    )


def _recombine(acc_pairs, denp_ref, denn_ref, q_ref, b_ref):
    q = q_ref[0, 0, 0, :]
    q2 = q_ref[1, 0, 0, :]
    den = (q * jnp.sum(denp_ref[...], axis=0)
           + q2 * jnp.sum(denn_ref[...], axis=0))
    inv = 1.0 / (den + 1e-16)
    parts = []
    for ap, an in acc_pairs:
        pos = ap[0].astype(jnp.float32) + ap[1].astype(jnp.float32)
        neg = an[0].astype(jnp.float32) + an[1].astype(jnp.float32)
        parts.append(q[:, None] * pos + q2[:, None] * neg)
    h = jnp.concatenate(parts, axis=1) * inv[:, None] + b_ref[...][None, :]
    return jnp.where(h >= 0.0, h, 0.01 * h)


def _make_tc_midprep(F_in, F_out):
    nq_in = F_in // FQ
    nq = F_out // FQ

    def body(*args):
        a_flat = args[:2 * nq_in]
        a_pairs = [(a_flat[2 * k], a_flat[2 * k + 1]) for k in range(nq_in)]
        (denp_ref, denn_ref, q1_ref, b_ref,
         w_ref, al_ref, ar_ref) = args[2 * nq_in:2 * nq_in + 7]
        rest = args[2 * nq_in + 7:]
        g_refs = rest[:nq]
        q_ref, el_ref, er_ref, feat_s, el_s, er_s, m_s = rest[nq:]
        i = pl.program_id(0)
        i0 = i % NB

        def feat_new():
            h = _recombine(a_pairs, denp_ref, denn_ref, q1_ref, b_ref)
            return jnp.dot(h, w_ref[...], preferred_element_type=jnp.float32)

        _featprep_tail(i, i0, feat_new, al_ref, ar_ref, g_refs, q_ref,
                       el_ref, er_ref, feat_s, el_s, er_s, m_s)

    return pl.pallas_call(
        body,
        grid=(2 * NB,),
        in_specs=(
            [pl.BlockSpec((NC, BLK, FQ), lambda i: (0, i % NB, 0)),
             pl.BlockSpec((NC, BLK, FQ), lambda i: (0, NB + i % NB, 0))]
            * nq_in
            + [
                pl.BlockSpec((NW, BLK), lambda i: (0, i % NB)),
                pl.BlockSpec((NW, BLK), lambda i: (0, NB + i % NB)),
                pl.BlockSpec((2, 1, 1, BLK), lambda i: (0, i % NB, 0, 0)),
                pl.BlockSpec((F_in,), lambda i: (0,)),
                pl.BlockSpec((F_in, F_out), lambda i: (0, 0)),
                pl.BlockSpec((F_out,), lambda i: (0,)),
                pl.BlockSpec((F_out,), lambda i: (0,)),
            ]
        ),
        out_specs=(
            [pl.BlockSpec((2, BLK, FQ), lambda i: (0, i % NB, 0))] * nq
            + [
                pl.BlockSpec((2, 1, 1, BLK), lambda i: (0, i % NB, 0, 0)),
                pl.BlockSpec((BLK,), lambda i: (i % NB,)),
                pl.BlockSpec((BLK,), lambda i: (i % NB,)),
            ]
        ),
        out_shape=(
            [jax.ShapeDtypeStruct((2, N_PAD, FQ), jnp.bfloat16)] * nq
            + [
                jax.ShapeDtypeStruct((2, NB, 1, BLK), jnp.float32),
                jax.ShapeDtypeStruct((N_PAD,), jnp.float32),
                jax.ShapeDtypeStruct((N_PAD,), jnp.float32),
            ]
        ),
        scratch_shapes=[
            pltpu.VMEM((N_PAD, F_out), jnp.float32),
            pltpu.VMEM((NB, BLK), jnp.float32),
            pltpu.VMEM((NB, BLK), jnp.float32),
            pltpu.SMEM((1, 2), jnp.float32),
        ],
    )


def _final_body(a0p, a0n, denp_ref, denn_ref, q_ref, b_ref, out_ref):
    i = pl.program_id(0)
    h = _recombine(((a0p, a0n),), denp_ref, denn_ref, q_ref, b_ref)
    row = i * BLK + lax.broadcasted_iota(jnp.int32, (BLK, 1), 0)
    h = jnp.where(row < N, h, 0.0)

    @pl.when(i == 0)
    def _():
        out_ref[...] = jnp.zeros_like(out_ref)

    out_ref[...] += jnp.sum(h, axis=0, keepdims=True)

    @pl.when(i == NB - 1)
    def _():
        out_ref[...] *= jnp.float32(1.0 / N)


def _make_tc_final(F):
    nq = F // FQ
    return pl.pallas_call(
        _final_body,
        grid=(NB,),
        in_specs=(
            [pl.BlockSpec((NC, BLK, FQ), lambda i: (0, i, 0)),
             pl.BlockSpec((NC, BLK, FQ), lambda i: (0, NB + i, 0))] * nq
            + [
                pl.BlockSpec((NW, BLK), lambda i: (0, i)),
                pl.BlockSpec((NW, BLK), lambda i: (0, NB + i)),
                pl.BlockSpec((2, 1, 1, BLK), lambda i: (0, i, 0, 0)),
                pl.BlockSpec((F,), lambda i: (0,)),
            ]
        ),
        out_specs=pl.BlockSpec((1, F), lambda i: (0, 0)),
        out_shape=jax.ShapeDtypeStruct((1, F), jnp.float32),
    )


def _make_sc_edge(nparts):
    mesh = plsc.VectorSubcoreMesh(core_axis_name="c", subcore_axis_name="s")

    @functools.partial(
        pl.kernel,
        out_type=(
            [jax.ShapeDtypeStruct((NC, N2, FQ), jnp.bfloat16)] * nparts
            + [jax.ShapeDtypeStruct((NW, N2), jnp.float32)]
        ),
        mesh=mesh,
        compiler_params=pltpu.CompilerParams(
            needs_layout_passes=False, use_tc_tiling_on_sc=False),
        scratch_types=[
            pltpu.VMEM((EPW,), jnp.int32),
            pltpu.VMEM((EPW,), jnp.int32),
            pltpu.VMEM((N_PAD,), jnp.float32),
            pltpu.VMEM((N_PAD,), jnp.float32),
            pltpu.VMEM((N2,), jnp.float32),
            pltpu.VMEM((4, C, FQ), jnp.bfloat16),
            pltpu.SemaphoreType.DMA,
            pltpu.SemaphoreType.DMA,
            pltpu.SemaphoreType.DMA,
            pltpu.SemaphoreType.DMA,
            pltpu.SemaphoreType.DMA,
            pltpu.SemaphoreType.DMA,
            pltpu.SemaphoreType.DMA,
            pltpu.SemaphoreType.DMA,
            pltpu.VMEM_SHARED((N2, FQ), jnp.bfloat16),
        ],
    )
    def sc_edge(*args):
        (ei_h, el_h, er_h) = args[:3]
        g_hs = args[3:3 + nparts]
        agg_outs = args[3 + nparts:3 + 2 * nparts]
        den_out = args[3 + 2 * nparts]
        (src1, dst1, el_v, er_v, den_v, rows_v,
         g0, g1, g2, g3, s0, s1, s2, s3, agg_sh) = args[4 + 2 * nparts:]
        gsems = (g0, g1, g2, g3)
        ssems = (s0, s1, s2, s3)

        cid = lax.axis_index("c")
        sid = lax.axis_index("s")
        wid = cid * NS + sid
        base = sid * RPT2

        pltpu.sync_copy(ei_h.at[0, pl.ds(wid * EPW, EPW)], src1)
        pltpu.sync_copy(ei_h.at[1, pl.ds(wid * EPW, EPW)], dst1)
        pltpu.sync_copy(el_h, el_v)
        pltpu.sync_copy(er_h, er_v)

        zvec = jnp.zeros((L,), jnp.float32)
        zvec_b = jnp.zeros((2 * L,), jnp.bfloat16)

        def _zero_acc_slice():
            def zrow(r, carry):
                rows_v[0, r, pl.ds(0, 2 * L)] = zvec_b
                rows_v[0, r, pl.ds(2 * L, 2 * L)] = zvec_b
                return carry

            lax.fori_loop(0, C, zrow, 0)
            for t in range(RPT2 // C):
                pltpu.sync_copy(rows_v.at[0],
                                agg_sh.at[pl.ds(base + t * C, C)])

        _zero_acc_slice()

        def zden(r, carry):
            den_v[pl.ds(r * L, L)] = zvec
            return carry

        lax.fori_loop(0, DROW2, zden, 0)

        def passa(t, carry):
            s_idx = src1[pl.ds(t * L, L)]
            d_idx = dst1[pl.ds(t * L, L)]
            els = plsc.load_gather(el_v, [s_idx])
            e = els + plsc.load_gather(er_v, [d_idx])
            neg = e < 0.0
            adj = jnp.where(neg, N_PAD, 0).astype(jnp.int32)
            si = s_idx + adj
            di = d_idx + adj
            src1[pl.ds(t * L, L)] = si
            dst1[pl.ds(t * L, L)] = di
            v = jnp.exp(jnp.where(neg, 0.2 * els, els))
            plsc.addupdate_scatter(den_v, [di], v)
            return carry

        lax.fori_loop(0, EPW // L, passa, 0)
        pltpu.sync_copy(den_v, den_out.at[wid])

        for p in range(nparts):
            plsc.subcore_barrier()

            g_h = g_hs[p]
            for b in range(4):
                pltpu.async_copy(g_h.at[src1.at[pl.ds(b * C, C)]],
                                 rows_v.at[b], gsems[b])

            def passb(i, carry):
                j4 = 4 * i
                for b in range(4):
                    j = j4 + b
                    pltpu.make_async_copy(
                        g_h.at[src1.at[pl.ds(j * C, C)]],
                        rows_v.at[b], gsems[b]).wait()
                    pltpu.async_copy(
                        rows_v.at[b], agg_sh.at[dst1.at[pl.ds(j * C, C)]],
                        ssems[b], add=True)
                for b in range(4):
                    j = j4 + b
                    pltpu.make_async_copy(
                        rows_v.at[b], agg_sh.at[dst1.at[pl.ds(j * C, C)]],
                        ssems[b]).wait()

                    @pl.when(j + 4 < NCHUNK)
                    def _():
                        pltpu.async_copy(
                            g_h.at[src1.at[pl.ds((j + 4) * C, C)]],
                            rows_v.at[b], gsems[b])
                return carry

            lax.fori_loop(0, NCHUNK // 4, passb, 0)
            jt = NCHUNK - 1
            pltpu.make_async_copy(
                g_h.at[src1.at[pl.ds(jt * C, C)]], rows_v.at[0], gsems[0]).wait()
            pltpu.sync_copy(rows_v.at[0],
                            agg_sh.at[dst1.at[pl.ds(jt * C, C)]], add=True)

            plsc.subcore_barrier()

            pltpu.sync_copy(agg_sh.at[pl.ds(base, RPT2)],
                            agg_outs[p].at[cid, pl.ds(base, RPT2)])
            if p + 1 < nparts:
                _zero_acc_slice()

    return sc_edge


_tc_featprep1 = _make_tc_featprep(IN_F, H1_F)
_tc_midprep = _make_tc_midprep(H1_F, OUT_F)
_tc_final = _make_tc_final(OUT_F)
_sc_edge1 = _make_sc_edge(H1_F // FQ)
_sc_edge2 = _make_sc_edge(OUT_F // FQ)


def kernel(x, edge_index, W1, attn_l1, attn_r1, b1, W2, attn_l2, attn_r2, b2):
    x_pad = jnp.pad(x, ((0, N_PAD - N), (0, 0)))

    g10, g11, q1, el1, er1 = _tc_featprep1(x_pad, W1, attn_l1, attn_r1)
    a0, a1, den1 = _sc_edge1(
        edge_index, el1, er1, g10.reshape(N2, FQ), g11.reshape(N2, FQ))

    g20, q2, el2, er2 = _tc_midprep(
        a0, a0, a1, a1, den1, den1, q1, b1, W2, attn_l2, attn_r2)
    c0, den2 = _sc_edge2(edge_index, el2, er2, g20.reshape(N2, FQ))

    return _tc_final(c0, c0, den2, den2, q2, b2)
